# Initial kernel scaffold; baseline (speedup 1.0000x reference)
import jax
import jax.numpy as jnp
from jax.experimental import pallas as pl


def kernel(x, w1_1, w1_2, w1_3, w2_1, w2_2, w2_3, w3, bn1_1_gamma, bn1_1_beta, bn1_1_mean, bn1_1_var, bn1_2_gamma, bn1_2_beta, bn1_2_mean, bn1_2_var, bn1_3_gamma, bn1_3_beta, bn1_3_mean, bn1_3_var, bn2_1_gamma, bn2_1_beta, bn2_1_mean, bn2_1_var, bn2_2_gamma, bn2_2_beta, bn2_2_mean, bn2_2_var, bn2_3_gamma, bn2_3_beta, bn2_3_mean, bn2_3_var):
    raise NotImplementedError("write your pallas kernel here")


# speedup vs baseline: 1.5391x; 1.5391x over previous
from functools import partial

import jax
import jax.numpy as jnp
from jax.experimental import pallas as pl
from jax.experimental.pallas import tpu as pltpu

_EPS = 1e-5

_G1 = 28
_G2 = 16
_S1 = _G1 * _G1

_R1 = 726
_R2 = 668
_RP = 192
_R4 = 154
_R5 = 120
_R6 = 86
_BANDS = 12
_HEXT = 6

_NB = 16


def _taps_sum(src, w_ref, k, grid_w, out_rows):
    acc = None
    for kh in range(k):
        for kw in range(k):
            xt = src[pl.ds(kh * grid_w + kw, out_rows), :]
            y = jnp.dot(xt, w_ref[kh * k + kw],
                        preferred_element_type=jnp.float32)
            acc = y if acc is None else acc + y
    return acc


def _net_kernel(x_ref,
                w1, w2, c2, w3c, c3, s3, b3,
                w4, w5, c5, w6, c6, wh, bh,
                o_ref, t1, t3, tp, t4, t5, tf):
    t1[...] = jnp.maximum(_taps_sum(x_ref, w1, 3, _G1, _R1), 0.0)

    y2 = jnp.maximum(_taps_sum(t1, w2, 3, _G1, _R2) + c2[...], 0.0)

    z3 = jnp.dot(y2, w3c[0], preferred_element_type=jnp.float32)
    t3[...] = jnp.maximum(z3 + c3[...], 0.0) * s3[...] + b3[...]

    t1[pl.ds(0, _R2 - 1), :] = jnp.maximum(t3[pl.ds(0, _R2 - 1), :],
                                           t3[pl.ds(1, _R2 - 1), :])
    tp[...] = jnp.zeros_like(tp)
    for i in range(_BANDS):
        base = 2 * i * _G1
        tp[pl.ds(_G2 * i, _BANDS), :] = jnp.maximum(
            t1[pl.ds(base, _BANDS, 2), :],
            t1[pl.ds(base + _G1, _BANDS, 2), :])

    t4[...] = jnp.maximum(_taps_sum(tp, w4, 3, _G2, _R4), 0.0)

    t5[...] = jnp.maximum(_taps_sum(t4, w5, 3, _G2, _R5) + c5[...], 0.0)

    r6 = jnp.maximum(_taps_sum(t5, w6, 3, _G2, _R6) + c6[...], 0.0)

    ssum = None
    for i in range(_HEXT):
        band = jnp.sum(r6[_G2 * i:_G2 * i + _HEXT, :], axis=0, keepdims=True)
        ssum = band if ssum is None else ssum + band
    pooled = ssum * (1.0 / (_HEXT * _HEXT))
    for b in range(_NB):
        tf[pl.ds(b, 1), :] = pooled[:, 16 * b:16 * (b + 1)]
    logits = jnp.dot(tf[...], wh[...],
                     preferred_element_type=jnp.float32) + bh[...]
    zc = logits - jnp.max(logits, axis=-1, keepdims=True)
    o_ref[...] = zc - jnp.log(jnp.sum(jnp.exp(zc), axis=-1, keepdims=True))


def _fold(gamma, beta, mean, var):
    scale = gamma / jnp.sqrt(var + _EPS)
    return scale, beta - mean * scale


def _block_taps(w, scale_in=None):
    w = w.astype(jnp.float32)
    if scale_in is not None:
        w = w * scale_in.astype(jnp.float32)[None, :, None, None]
    cout, cin = w.shape[0], w.shape[1]
    taps = jnp.transpose(w, (2, 3, 1, 0)).reshape(-1, cin, cout)
    eye = jnp.eye(_NB, dtype=jnp.float32)
    wbd = jnp.einsum("ab,tio->taibo", eye, taps)
    return wbd.reshape(taps.shape[0], _NB * cin, _NB * cout)


def _shift_bias(w, shift_in):
    c = jnp.einsum("oikl,i->o", w.astype(jnp.float32),
                   shift_in.astype(jnp.float32))
    return jnp.tile(c, _NB).reshape(1, -1)


def kernel(x, w1_1, w1_2, w1_3, w2_1, w2_2, w2_3, w3,
           bn1_1_gamma, bn1_1_beta, bn1_1_mean, bn1_1_var,
           bn1_2_gamma, bn1_2_beta, bn1_2_mean, bn1_2_var,
           bn1_3_gamma, bn1_3_beta, bn1_3_mean, bn1_3_var,
           bn2_1_gamma, bn2_1_beta, bn2_1_mean, bn2_1_var,
           bn2_2_gamma, bn2_2_beta, bn2_2_mean, bn2_2_var,
           bn2_3_gamma, bn2_3_beta, bn2_3_mean, bn2_3_var):
    s1, sh1 = _fold(bn1_1_gamma, bn1_1_beta, bn1_1_mean, bn1_1_var)
    s2, sh2 = _fold(bn1_2_gamma, bn1_2_beta, bn1_2_mean, bn1_2_var)
    s3, sh3 = _fold(bn1_3_gamma, bn1_3_beta, bn1_3_mean, bn1_3_var)
    s4, sh4 = _fold(bn2_1_gamma, bn2_1_beta, bn2_1_mean, bn2_1_var)
    s5, sh5 = _fold(bn2_2_gamma, bn2_2_beta, bn2_2_mean, bn2_2_var)
    s6, sh6 = _fold(bn2_3_gamma, bn2_3_beta, bn2_3_mean, bn2_3_var)

    n = x.shape[0]
    steps = -(-n // _NB)
    n_pad = steps * _NB
    xf = x.astype(jnp.float32).reshape(n, _S1)
    if n_pad != n:
        xf = jnp.concatenate(
            [xf, jnp.zeros((n_pad - n, _S1), jnp.float32)], axis=0)
    xs = xf.reshape(steps, _NB, _S1).transpose(0, 2, 1)

    w2b = _block_taps(w1_2, scale_in=s1)
    c2 = _shift_bias(w1_2, sh1)
    w3b = _block_taps(w1_3, scale_in=s2)
    c3 = _shift_bias(w1_3, sh2)
    w5b = _block_taps(w2_2, scale_in=s4)
    c5 = _shift_bias(w2_2, sh4)
    w6b = _block_taps(w2_3, scale_in=s5)
    c6 = _shift_bias(w2_3, sh5)
    s3t = jnp.tile(s3, _NB).reshape(1, -1)
    b3t = jnp.tile(sh3, _NB).reshape(1, -1)
    w1b = _block_taps(w1_1)
    w4b = _block_taps(w2_1)
    w3f = jnp.transpose(w3[:, :, 0, 0]).astype(jnp.float32)
    wh = w3f * s6.astype(jnp.float32)[:, None]
    bh = (sh6.astype(jnp.float32) @ w3f).reshape(1, 10)

    full = lambda *shape: pl.BlockSpec(shape, lambda s: (0,) * len(shape))
    in_specs = [
        pl.BlockSpec((None, _S1, _NB), lambda s: (s, 0, 0)),
        full(9, _NB * 1, _NB * 8),
        full(9, _NB * 8, _NB * 16),
        full(1, _NB * 16),
        full(1, _NB * 16, _NB * 8),
        full(1, _NB * 8),
        full(1, _NB * 8),
        full(1, _NB * 8),
        full(9, _NB * 8, _NB * 12),
        full(9, _NB * 12, _NB * 16),
        full(1, _NB * 16),
        full(9, _NB * 16, _NB * 16),
        full(1, _NB * 16),
        full(16, 10),
        full(1, 10),
    ]
    args = (xs, w1b, w2b, c2, w3b, c3, s3t, b3t,
            w4b, w5b, c5, w6b, c6, wh, bh)

    out = pl.pallas_call(
        _net_kernel,
        out_shape=jax.ShapeDtypeStruct((steps, _NB, 10), jnp.float32),
        grid=(steps,),
        in_specs=in_specs,
        out_specs=pl.BlockSpec((None, _NB, 10), lambda s: (s, 0, 0)),
        scratch_shapes=[
            pltpu.VMEM((_R1, _NB * 8), jnp.float32),
            pltpu.VMEM((_R2, _NB * 8), jnp.float32),
            pltpu.VMEM((_RP, _NB * 8), jnp.float32),
            pltpu.VMEM((_R4, _NB * 12), jnp.float32),
            pltpu.VMEM((_R5, _NB * 16), jnp.float32),
            pltpu.VMEM((_NB, 16), jnp.float32),
        ],
        compiler_params=pltpu.CompilerParams(
            dimension_semantics=("parallel",),
            vmem_limit_bytes=48 * 1024 * 1024,
        ),
    )(*args)
    return out.reshape(n_pad, 10)[:n]


# speedup vs baseline: 2.3175x; 1.5057x over previous
import jax
import jax.numpy as jnp
from jax.experimental import pallas as pl
from jax.experimental.pallas import tpu as pltpu

_EPS = 1e-5

_G1 = 28
_G2 = 16
_S1 = _G1 * _G1

_R1 = 728
_R2 = 672
_RP = 192
_R4 = 154
_R5 = 120
_R6 = 86
_BANDS = 12
_HEXT = 6

_NB = 16
_L = _NB * 8


def _net_kernel(x_ref,
                w1k, w2p, c2, w3c, c3, s3, b3,
                w4p, w5, c5, w6, c6, wh, bh,
                o_ref, x3, ca, cb, t3, ts, pa, pb, t4, t5, tf):
    f32 = jnp.float32

    xv = x_ref[...]
    z1r = jnp.zeros((1, _NB), f32)
    x3[...] = jnp.concatenate(
        [xv,
         jnp.concatenate([xv[1:], z1r], axis=0),
         jnp.concatenate([xv[2:], z1r, z1r], axis=0),
         jnp.zeros((_S1, _L - 48), f32)], axis=1)

    z1 = (jnp.dot(x3[pl.ds(0, _R1), :], w1k[0], preferred_element_type=f32)
          + jnp.dot(x3[pl.ds(_G1, _R1), :], w1k[1], preferred_element_type=f32)
          + jnp.dot(x3[pl.ds(2 * _G1, _R1), :], w1k[2],
                    preferred_element_type=f32))
    v1 = jnp.maximum(z1, 0.0)

    ca[...] = jnp.concatenate(
        [v1, jnp.concatenate([v1[1:], v1[:1]], axis=0)], axis=1)
    cb[...] = jnp.concatenate(
        [v1, jnp.concatenate([v1[_G1:], v1[:_G1]], axis=0)], axis=1)

    z2 = (jnp.dot(ca[pl.ds(0, _R2), :], w2p[0], preferred_element_type=f32)
          + jnp.dot(ca[pl.ds(_G1, _R2), :], w2p[1], preferred_element_type=f32)
          + jnp.dot(ca[pl.ds(2 * _G1, _R2), :], w2p[2],
                    preferred_element_type=f32)
          + jnp.dot(cb[pl.ds(2, _R2), :], w2p[3], preferred_element_type=f32)
          + jnp.dot(cb[pl.ds(_G1 + 2, _R2), :], w2p[4],
                    preferred_element_type=f32))
    y2 = jnp.maximum(z2 + c2[...], 0.0)

    z3 = jnp.dot(y2, w3c[0], preferred_element_type=f32)
    t3[...] = jnp.maximum(z3 + c3[...], 0.0) * s3[...] + b3[...]

    ts[pl.ds(0, _R2 - 1), :] = jnp.maximum(t3[pl.ds(0, _R2 - 1), :],
                                           t3[pl.ds(1, _R2 - 1), :])
    prev = None
    for i in range(_BANDS):
        base = 2 * i * _G1
        p = jnp.maximum(ts[pl.ds(base, _BANDS, 2), :],
                        ts[pl.ds(base + _G1, _BANDS, 2), :])
        pa[pl.ds(_G2 * i, _BANDS), :] = jnp.concatenate(
            [p, jnp.concatenate([p[1:], p[:1]], axis=0)], axis=1)
        if prev is not None:
            pb[pl.ds(_G2 * (i - 1), _BANDS), :] = jnp.concatenate(
                [prev, p], axis=1)
        prev = p
    pb[pl.ds(_G2 * (_BANDS - 1), _BANDS), :] = jnp.concatenate(
        [prev, prev], axis=1)

    z4 = (jnp.dot(pa[pl.ds(0, _R4), :], w4p[0], preferred_element_type=f32)
          + jnp.dot(pa[pl.ds(_G2, _R4), :], w4p[1], preferred_element_type=f32)
          + jnp.dot(pa[pl.ds(2 * _G2, _R4), :], w4p[2],
                    preferred_element_type=f32)
          + jnp.dot(pb[pl.ds(2, _R4), :], w4p[3], preferred_element_type=f32)
          + jnp.dot(pb[pl.ds(_G2 + 2, _R4), :], w4p[4],
                    preferred_element_type=f32))
    t4[...] = jnp.maximum(z4, 0.0)

    z5 = None
    for kh in range(3):
        for kw in range(3):
            y = jnp.dot(t4[pl.ds(kh * _G2 + kw, _R5), :], w5[3 * kh + kw],
                        preferred_element_type=f32)
            z5 = y if z5 is None else z5 + y
    t5[...] = jnp.maximum(z5 + c5[...], 0.0)

    z6 = None
    for kh in range(3):
        for kw in range(3):
            y = jnp.dot(t5[pl.ds(kh * _G2 + kw, _R6), :], w6[3 * kh + kw],
                        preferred_element_type=f32)
            z6 = y if z6 is None else z6 + y
    r6 = jnp.maximum(z6 + c6[...], 0.0)

    ssum = None
    for i in range(_HEXT):
        band = jnp.sum(r6[_G2 * i:_G2 * i + _HEXT, :], axis=0, keepdims=True)
        ssum = band if ssum is None else ssum + band
    pooled = ssum * (1.0 / (_HEXT * _HEXT))
    for b in range(_NB):
        tf[pl.ds(b, 1), :] = pooled[:, 16 * b:16 * (b + 1)]
    logits = jnp.dot(tf[...], wh[...], preferred_element_type=f32) + bh[...]
    zc = logits - jnp.max(logits, axis=-1, keepdims=True)
    o_ref[...] = zc - jnp.log(jnp.sum(jnp.exp(zc), axis=-1, keepdims=True))


def _fold(gamma, beta, mean, var):
    scale = gamma / jnp.sqrt(var + _EPS)
    return scale, beta - mean * scale


def _block_taps(w, scale_in=None):
    w = w.astype(jnp.float32)
    if scale_in is not None:
        w = w * scale_in.astype(jnp.float32)[None, :, None, None]
    cout, cin = w.shape[0], w.shape[1]
    taps = jnp.transpose(w, (2, 3, 1, 0)).reshape(-1, cin, cout)
    eye = jnp.eye(_NB, dtype=jnp.float32)
    wbd = jnp.einsum("ab,tio->taibo", eye, taps)
    return wbd.reshape(taps.shape[0], _NB * cin, _NB * cout)


def _pair_w(taps):
    k, n = taps.shape[1], taps.shape[2]
    z = jnp.zeros((k, n), jnp.float32)
    cat = lambda a, b: jnp.concatenate([a, b], axis=0)
    return jnp.stack([cat(taps[0], taps[1]), cat(taps[3], taps[4]),
                      cat(taps[6], taps[7]), cat(taps[2], taps[5]),
                      cat(z, taps[8])])


def _shift_bias(w, shift_in):
    c = jnp.einsum("oikl,i->o", w.astype(jnp.float32),
                   shift_in.astype(jnp.float32))
    return jnp.tile(c, _NB).reshape(1, -1)


def kernel(x, w1_1, w1_2, w1_3, w2_1, w2_2, w2_3, w3,
           bn1_1_gamma, bn1_1_beta, bn1_1_mean, bn1_1_var,
           bn1_2_gamma, bn1_2_beta, bn1_2_mean, bn1_2_var,
           bn1_3_gamma, bn1_3_beta, bn1_3_mean, bn1_3_var,
           bn2_1_gamma, bn2_1_beta, bn2_1_mean, bn2_1_var,
           bn2_2_gamma, bn2_2_beta, bn2_2_mean, bn2_2_var,
           bn2_3_gamma, bn2_3_beta, bn2_3_mean, bn2_3_var):
    s1, sh1 = _fold(bn1_1_gamma, bn1_1_beta, bn1_1_mean, bn1_1_var)
    s2, sh2 = _fold(bn1_2_gamma, bn1_2_beta, bn1_2_mean, bn1_2_var)
    s3, sh3 = _fold(bn1_3_gamma, bn1_3_beta, bn1_3_mean, bn1_3_var)
    s4, sh4 = _fold(bn2_1_gamma, bn2_1_beta, bn2_1_mean, bn2_1_var)
    s5, sh5 = _fold(bn2_2_gamma, bn2_2_beta, bn2_2_mean, bn2_2_var)
    s6, sh6 = _fold(bn2_3_gamma, bn2_3_beta, bn2_3_mean, bn2_3_var)

    n = x.shape[0]
    steps = -(-n // _NB)
    n_pad = steps * _NB
    xf = x.astype(jnp.float32).reshape(n, _S1)
    if n_pad != n:
        xf = jnp.concatenate(
            [xf, jnp.zeros((n_pad - n, _S1), jnp.float32)], axis=0)
    xs = xf.reshape(steps, _NB, _S1).transpose(0, 2, 1)

    t1aps = _block_taps(w1_1)
    kpad = jnp.zeros((_L - 48, _L), jnp.float32)
    w1k = jnp.stack([jnp.concatenate([t1aps[3 * kh], t1aps[3 * kh + 1],
                                      t1aps[3 * kh + 2], kpad], axis=0)
                     for kh in range(3)])
    w2p = _pair_w(_block_taps(w1_2, scale_in=s1))
    c2 = _shift_bias(w1_2, sh1)
    w3b = _block_taps(w1_3, scale_in=s2)
    c3 = _shift_bias(w1_3, sh2)
    w4p = _pair_w(_block_taps(w2_1))
    w5b = _block_taps(w2_2, scale_in=s4)
    c5 = _shift_bias(w2_2, sh4)
    w6b = _block_taps(w2_3, scale_in=s5)
    c6 = _shift_bias(w2_3, sh5)
    s3t = jnp.tile(s3, _NB).reshape(1, -1)
    b3t = jnp.tile(sh3, _NB).reshape(1, -1)
    w3f = jnp.transpose(w3[:, :, 0, 0]).astype(jnp.float32)
    wh = w3f * s6.astype(jnp.float32)[:, None]
    bh = (sh6.astype(jnp.float32) @ w3f).reshape(1, 10)

    full = lambda *shape: pl.BlockSpec(shape, lambda s: (0,) * len(shape))
    in_specs = [
        pl.BlockSpec((None, _S1, _NB), lambda s: (s, 0, 0)),
        full(3, _L, _L),
        full(5, 256, 256),
        full(1, 256),
        full(1, 256, _L),
        full(1, _L),
        full(1, _L),
        full(1, _L),
        full(5, 256, 192),
        full(9, 192, 256),
        full(1, 256),
        full(9, 256, 256),
        full(1, 256),
        full(16, 10),
        full(1, 10),
    ]
    args = (xs, w1k, w2p, c2, w3b, c3, s3t, b3t,
            w4p, w5b, c5, w6b, c6, wh, bh)

    out = pl.pallas_call(
        _net_kernel,
        out_shape=jax.ShapeDtypeStruct((steps, _NB, 10), jnp.float32),
        grid=(steps,),
        in_specs=in_specs,
        out_specs=pl.BlockSpec((None, _NB, 10), lambda s: (s, 0, 0)),
        scratch_shapes=[
            pltpu.VMEM((_S1, _L), jnp.float32),
            pltpu.VMEM((_R1, 2 * _L), jnp.float32),
            pltpu.VMEM((_R1, 2 * _L), jnp.float32),
            pltpu.VMEM((_R2, _L), jnp.float32),
            pltpu.VMEM((_R2, _L), jnp.float32),
            pltpu.VMEM((_RP, 2 * _L), jnp.float32),
            pltpu.VMEM((_RP, 2 * _L), jnp.float32),
            pltpu.VMEM((_R4, _NB * 12), jnp.float32),
            pltpu.VMEM((_R5, _NB * 16), jnp.float32),
            pltpu.VMEM((_NB, 16), jnp.float32),
        ],
        compiler_params=pltpu.CompilerParams(
            dimension_semantics=("parallel",),
            vmem_limit_bytes=48 * 1024 * 1024,
        ),
    )(*args)
    return out.reshape(n_pad, 10)[:n]


# speedup vs baseline: 2.3799x; 1.0269x over previous
import jax
import jax.numpy as jnp
from jax.experimental import pallas as pl
from jax.experimental.pallas import tpu as pltpu

_EPS = 1e-5

_G1 = 28
_G2 = 16
_S1 = _G1 * _G1

_R1 = 728
_R2 = 672
_RP = 192
_R4 = 154
_R5 = 120
_R6 = 86
_BANDS = 12
_HEXT = 6

_NB = 16
_L = _NB * 8


def _net_kernel(x_ref,
                w1k, w2p, c2, w3c, c3, s3, b3,
                w4p, w5, c5, w6, c6, wh, bh,
                o_ref, x3, ca, cb, t3, ts, pa, pb, t4, t5, tf):
    f32 = jnp.float32
    bf16 = jnp.bfloat16

    xv = jnp.transpose(x_ref[...]).astype(bf16)
    z1r = jnp.zeros((1, _NB), bf16)
    x3[...] = jnp.concatenate(
        [xv,
         jnp.concatenate([xv[1:], z1r], axis=0),
         jnp.concatenate([xv[2:], z1r, z1r], axis=0),
         jnp.zeros((_S1, _L - 48), bf16)], axis=1)

    z1 = (jnp.dot(x3[pl.ds(0, _R1), :], w1k[0], preferred_element_type=f32)
          + jnp.dot(x3[pl.ds(_G1, _R1), :], w1k[1], preferred_element_type=f32)
          + jnp.dot(x3[pl.ds(2 * _G1, _R1), :], w1k[2],
                    preferred_element_type=f32))
    v1 = jnp.maximum(z1, 0.0)

    v1h = v1.astype(bf16)
    ca[...] = jnp.concatenate(
        [v1h, jnp.concatenate([v1h[1:], v1h[:1]], axis=0)], axis=1)
    cb[...] = jnp.concatenate(
        [v1h, jnp.concatenate([v1h[_G1:], v1h[:_G1]], axis=0)], axis=1)

    d0 = jnp.dot(ca[pl.ds(0, _R2), :], w2p[0], preferred_element_type=f32)
    d1 = jnp.dot(ca[pl.ds(_G1, _R2), :], w2p[1], preferred_element_type=f32)
    d2 = jnp.dot(ca[pl.ds(2 * _G1, _R2), :], w2p[2],
                 preferred_element_type=f32)
    d3 = jnp.dot(cb[pl.ds(2, _R2), :], w2p[3], preferred_element_type=f32)
    d4 = jnp.dot(cb[pl.ds(_G1 + 2, _R2), :], w2p[4],
                 preferred_element_type=f32)
    z2 = ((d0 + d1) + (d2 + d3)) + d4
    y2 = jnp.maximum(z2 + c2[...], 0.0).astype(bf16)

    z3 = jnp.dot(y2, w3c[0], preferred_element_type=f32)
    t3[...] = jnp.maximum(z3 + c3[...], 0.0) * s3[...] + b3[...]

    ts[pl.ds(0, _R2 - 1), :] = jnp.maximum(t3[pl.ds(0, _R2 - 1), :],
                                           t3[pl.ds(1, _R2 - 1), :])
    prev = None
    for i in range(_BANDS):
        base = 2 * i * _G1
        p = jnp.maximum(ts[pl.ds(base, _BANDS, 2), :],
                        ts[pl.ds(base + _G1, _BANDS, 2), :]).astype(bf16)
        pa[pl.ds(_G2 * i, _BANDS), :] = jnp.concatenate(
            [p, jnp.concatenate([p[1:], p[:1]], axis=0)], axis=1)
        if prev is not None:
            pb[pl.ds(_G2 * (i - 1), _BANDS), :] = jnp.concatenate(
                [prev, p], axis=1)
        prev = p
    pb[pl.ds(_G2 * (_BANDS - 1), _BANDS), :] = jnp.concatenate(
        [prev, prev], axis=1)

    e0 = jnp.dot(pa[pl.ds(0, _R4), :], w4p[0], preferred_element_type=f32)
    e1 = jnp.dot(pa[pl.ds(_G2, _R4), :], w4p[1], preferred_element_type=f32)
    e2 = jnp.dot(pa[pl.ds(2 * _G2, _R4), :], w4p[2],
                 preferred_element_type=f32)
    e3 = jnp.dot(pb[pl.ds(2, _R4), :], w4p[3], preferred_element_type=f32)
    e4 = jnp.dot(pb[pl.ds(_G2 + 2, _R4), :], w4p[4],
                 preferred_element_type=f32)
    z4 = ((e0 + e1) + (e2 + e3)) + e4
    t4[...] = jnp.maximum(z4, 0.0).astype(bf16)


    d5 = [jnp.dot(t4[pl.ds(kh * _G2 + kw, _R5), :], w5[3 * kh + kw],
                  preferred_element_type=f32)
          for kh in range(3) for kw in range(3)]
    z5 = (((d5[0] + d5[1]) + (d5[2] + d5[3]))
          + ((d5[4] + d5[5]) + (d5[6] + d5[7]))) + d5[8]
    t5[...] = jnp.maximum(z5 + c5[...], 0.0).astype(bf16)

    d6 = [jnp.dot(t5[pl.ds(kh * _G2 + kw, _R6), :], w6[3 * kh + kw],
                  preferred_element_type=f32)
          for kh in range(3) for kw in range(3)]
    z6 = (((d6[0] + d6[1]) + (d6[2] + d6[3]))
          + ((d6[4] + d6[5]) + (d6[6] + d6[7]))) + d6[8]
    r6 = jnp.maximum(z6 + c6[...], 0.0)

    ssum = None
    for i in range(_HEXT):
        band = jnp.sum(r6[_G2 * i:_G2 * i + _HEXT, :], axis=0, keepdims=True)
        ssum = band if ssum is None else ssum + band
    pooled = ssum * (1.0 / (_HEXT * _HEXT))
    for b in range(_NB):
        tf[pl.ds(b, 1), :] = pooled[:, 16 * b:16 * (b + 1)]
    logits = jnp.dot(tf[...], wh[...], preferred_element_type=f32) + bh[...]
    zc = logits - jnp.max(logits, axis=-1, keepdims=True)
    o_ref[...] = zc - jnp.log(jnp.sum(jnp.exp(zc), axis=-1, keepdims=True))


def _fold(gamma, beta, mean, var):
    scale = gamma / jnp.sqrt(var + _EPS)
    return scale, beta - mean * scale


def _block_taps(w, scale_in=None):
    w = w.astype(jnp.float32)
    if scale_in is not None:
        w = w * scale_in.astype(jnp.float32)[None, :, None, None]
    cout, cin = w.shape[0], w.shape[1]
    taps = jnp.transpose(w, (2, 3, 1, 0)).reshape(-1, cin, cout)
    eye = jnp.eye(_NB, dtype=jnp.float32)
    wbd = jnp.einsum("ab,tio->taibo", eye, taps)
    return wbd.reshape(taps.shape[0], _NB * cin, _NB * cout)


def _pair_w(taps):
    k, n = taps.shape[1], taps.shape[2]
    z = jnp.zeros((k, n), jnp.float32)
    cat = lambda a, b: jnp.concatenate([a, b], axis=0)
    return jnp.stack([cat(taps[0], taps[1]), cat(taps[3], taps[4]),
                      cat(taps[6], taps[7]), cat(taps[2], taps[5]),
                      cat(z, taps[8])])


def _shift_bias(w, shift_in):
    c = jnp.einsum("oikl,i->o", w.astype(jnp.float32),
                   shift_in.astype(jnp.float32))
    return jnp.tile(c, _NB).reshape(1, -1)


def kernel(x, w1_1, w1_2, w1_3, w2_1, w2_2, w2_3, w3,
           bn1_1_gamma, bn1_1_beta, bn1_1_mean, bn1_1_var,
           bn1_2_gamma, bn1_2_beta, bn1_2_mean, bn1_2_var,
           bn1_3_gamma, bn1_3_beta, bn1_3_mean, bn1_3_var,
           bn2_1_gamma, bn2_1_beta, bn2_1_mean, bn2_1_var,
           bn2_2_gamma, bn2_2_beta, bn2_2_mean, bn2_2_var,
           bn2_3_gamma, bn2_3_beta, bn2_3_mean, bn2_3_var):
    s1, sh1 = _fold(bn1_1_gamma, bn1_1_beta, bn1_1_mean, bn1_1_var)
    s2, sh2 = _fold(bn1_2_gamma, bn1_2_beta, bn1_2_mean, bn1_2_var)
    s3, sh3 = _fold(bn1_3_gamma, bn1_3_beta, bn1_3_mean, bn1_3_var)
    s4, sh4 = _fold(bn2_1_gamma, bn2_1_beta, bn2_1_mean, bn2_1_var)
    s5, sh5 = _fold(bn2_2_gamma, bn2_2_beta, bn2_2_mean, bn2_2_var)
    s6, sh6 = _fold(bn2_3_gamma, bn2_3_beta, bn2_3_mean, bn2_3_var)

    n = x.shape[0]
    steps = -(-n // _NB)
    n_pad = steps * _NB
    xf = x.astype(jnp.float32).reshape(n, _S1)
    if n_pad != n:
        xf = jnp.concatenate(
            [xf, jnp.zeros((n_pad - n, _S1), jnp.float32)], axis=0)
    xs = xf.reshape(steps, _NB, _S1)

    bf16 = jnp.bfloat16
    t1aps = _block_taps(w1_1)
    kpad = jnp.zeros((_L - 48, _L), jnp.float32)
    w1k = jnp.stack([jnp.concatenate([t1aps[3 * kh], t1aps[3 * kh + 1],
                                      t1aps[3 * kh + 2], kpad], axis=0)
                     for kh in range(3)]).astype(bf16)
    w2p = _pair_w(_block_taps(w1_2, scale_in=s1)).astype(bf16)
    c2 = _shift_bias(w1_2, sh1)
    w3b = _block_taps(w1_3, scale_in=s2).astype(bf16)
    c3 = _shift_bias(w1_3, sh2)
    w21p = jnp.concatenate(
        [w2_1.astype(jnp.float32),
         jnp.zeros((4,) + w2_1.shape[1:], jnp.float32)], axis=0)
    w4p = _pair_w(_block_taps(w21p)).astype(bf16)
    w22p = jnp.concatenate(
        [w2_2.astype(jnp.float32),
         jnp.zeros((w2_2.shape[0], 4) + w2_2.shape[2:], jnp.float32)], axis=1)
    s4p = jnp.concatenate([s4, jnp.ones((4,), jnp.float32)])
    w5b = _block_taps(w22p, scale_in=s4p).astype(bf16)
    c5 = _shift_bias(w2_2, sh4)
    w6b = _block_taps(w2_3, scale_in=s5).astype(bf16)
    c6 = _shift_bias(w2_3, sh5)
    s3t = jnp.tile(s3, _NB).reshape(1, -1)
    b3t = jnp.tile(sh3, _NB).reshape(1, -1)
    w3f = jnp.transpose(w3[:, :, 0, 0]).astype(jnp.float32)
    wh = w3f * s6.astype(jnp.float32)[:, None]
    bh = (sh6.astype(jnp.float32) @ w3f).reshape(1, 10)

    full = lambda *shape: pl.BlockSpec(shape, lambda s: (0,) * len(shape))
    in_specs = [
        pl.BlockSpec((None, _NB, _S1), lambda s: (s, 0, 0)),
        full(3, _L, _L),
        full(5, 256, 256),
        full(1, 256),
        full(1, 256, _L),
        full(1, _L),
        full(1, _L),
        full(1, _L),
        full(5, 256, 256),
        full(9, 256, 256),
        full(1, 256),
        full(9, 256, 256),
        full(1, 256),
        full(16, 10),
        full(1, 10),
    ]
    args = (xs, w1k, w2p, c2, w3b, c3, s3t, b3t,
            w4p, w5b, c5, w6b, c6, wh, bh)

    out = pl.pallas_call(
        _net_kernel,
        out_shape=jax.ShapeDtypeStruct((steps, _NB, 10), jnp.float32),
        grid=(steps,),
        in_specs=in_specs,
        out_specs=pl.BlockSpec((None, _NB, 10), lambda s: (s, 0, 0)),
        scratch_shapes=[
            pltpu.VMEM((_S1, _L), jnp.bfloat16),
            pltpu.VMEM((_R1, 2 * _L), jnp.bfloat16),
            pltpu.VMEM((_R1, 2 * _L), jnp.bfloat16),
            pltpu.VMEM((_R2, _L), jnp.float32),
            pltpu.VMEM((_R2, _L), jnp.float32),
            pltpu.VMEM((_RP, 2 * _L), jnp.bfloat16),
            pltpu.VMEM((_RP, 2 * _L), jnp.bfloat16),
            pltpu.VMEM((_R4, _NB * 16), jnp.bfloat16),
            pltpu.VMEM((_R5, _NB * 16), jnp.bfloat16),
            pltpu.VMEM((_NB, 16), jnp.float32),
        ],
        compiler_params=pltpu.CompilerParams(
            dimension_semantics=("parallel",),
            vmem_limit_bytes=48 * 1024 * 1024,
        ),
    )(*args)
    return out.reshape(n_pad, 10)[:n]


# speedup vs baseline: 2.9964x; 1.2590x over previous
import jax
import jax.numpy as jnp
from jax.experimental import pallas as pl
from jax.experimental.pallas import tpu as pltpu

_EPS = 1e-5

_G1 = 28
_G2 = 16
_S1 = _G1 * _G1

_R1 = 728
_R2 = 672
_RP = 192
_R4 = 154
_R5 = 120
_R6 = 86
_BANDS = 12
_HEXT = 6

_NB = 16
_GRP = 2
_L = _NB * 8


def _net_kernel(x_ref,
                w1k, w2p, c2, w3c, c3, s3, b3,
                w4p, w5, c5, w6, c6, wh, bh,
                o_ref, *scr):
    f32 = jnp.float32
    bf16 = jnp.bfloat16
    S = [scr[10 * g:10 * (g + 1)] for g in range(_GRP)]

    def stage_x3(g):
        x3 = S[g][0]
        xv = jnp.transpose(x_ref[pl.ds(g * _NB, _NB), :]).astype(bf16)
        z1r = jnp.zeros((1, _NB), bf16)
        x3[...] = jnp.concatenate(
            [xv,
             jnp.concatenate([xv[1:], z1r], axis=0),
             jnp.concatenate([xv[2:], z1r, z1r], axis=0),
             jnp.zeros((_S1, _L - 48), bf16)], axis=1)

    def stage_conv1(g):
        x3, ca, cb = S[g][0], S[g][1], S[g][2]
        z1 = (jnp.dot(x3[pl.ds(0, _R1), :], w1k[0],
                      preferred_element_type=f32)
              + jnp.dot(x3[pl.ds(_G1, _R1), :], w1k[1],
                        preferred_element_type=f32)
              + jnp.dot(x3[pl.ds(2 * _G1, _R1), :], w1k[2],
                        preferred_element_type=f32))
        v1h = jnp.maximum(z1, 0.0).astype(bf16)
        ca[...] = jnp.concatenate(
            [v1h, jnp.concatenate([v1h[1:], v1h[:1]], axis=0)], axis=1)
        cb[...] = jnp.concatenate(
            [v1h, jnp.concatenate([v1h[_G1:], v1h[:_G1]], axis=0)], axis=1)

    def stage_conv23(g):
        ca, cb, t3 = S[g][1], S[g][2], S[g][3]
        d0 = jnp.dot(ca[pl.ds(0, _R2), :], w2p[0], preferred_element_type=f32)
        d1 = jnp.dot(ca[pl.ds(_G1, _R2), :], w2p[1],
                     preferred_element_type=f32)
        d2 = jnp.dot(ca[pl.ds(2 * _G1, _R2), :], w2p[2],
                     preferred_element_type=f32)
        d3 = jnp.dot(cb[pl.ds(2, _R2), :], w2p[3], preferred_element_type=f32)
        d4 = jnp.dot(cb[pl.ds(_G1 + 2, _R2), :], w2p[4],
                     preferred_element_type=f32)
        z2 = ((d0 + d1) + (d2 + d3)) + d4
        y2 = jnp.maximum(z2 + c2[...], 0.0).astype(bf16)
        z3 = jnp.dot(y2, w3c[0], preferred_element_type=f32)
        t3[...] = jnp.maximum(z3 + c3[...], 0.0) * s3[...] + b3[...]

    def stage_pool(g):
        t3, ts, pa, pb = S[g][3], S[g][4], S[g][5], S[g][6]
        ts[pl.ds(0, _R2 - 1), :] = jnp.maximum(t3[pl.ds(0, _R2 - 1), :],
                                               t3[pl.ds(1, _R2 - 1), :])
        prev = None
        for i in range(_BANDS):
            base = 2 * i * _G1
            p = jnp.maximum(ts[pl.ds(base, _BANDS, 2), :],
                            ts[pl.ds(base + _G1, _BANDS, 2), :]).astype(bf16)
            pa[pl.ds(_G2 * i, _BANDS), :] = jnp.concatenate(
                [p, jnp.concatenate([p[1:], p[:1]], axis=0)], axis=1)
            if prev is not None:
                pb[pl.ds(_G2 * (i - 1), _BANDS), :] = jnp.concatenate(
                    [prev, p], axis=1)
            prev = p
        pb[pl.ds(_G2 * (_BANDS - 1), _BANDS), :] = jnp.concatenate(
            [prev, prev], axis=1)

    def stage_conv4(g):
        pa, pb, t4 = S[g][5], S[g][6], S[g][7]
        e0 = jnp.dot(pa[pl.ds(0, _R4), :], w4p[0], preferred_element_type=f32)
        e1 = jnp.dot(pa[pl.ds(_G2, _R4), :], w4p[1],
                     preferred_element_type=f32)
        e2 = jnp.dot(pa[pl.ds(2 * _G2, _R4), :], w4p[2],
                     preferred_element_type=f32)
        e3 = jnp.dot(pb[pl.ds(2, _R4), :], w4p[3], preferred_element_type=f32)
        e4 = jnp.dot(pb[pl.ds(_G2 + 2, _R4), :], w4p[4],
                     preferred_element_type=f32)
        z4 = ((e0 + e1) + (e2 + e3)) + e4
        t4[...] = jnp.maximum(z4, 0.0).astype(bf16)

    def stage_conv5(g):
        t4, t5 = S[g][7], S[g][8]
        d5 = [jnp.dot(t4[pl.ds(kh * _G2 + kw, _R5), :], w5[3 * kh + kw],
                      preferred_element_type=f32)
              for kh in range(3) for kw in range(3)]
        z5 = (((d5[0] + d5[1]) + (d5[2] + d5[3]))
              + ((d5[4] + d5[5]) + (d5[6] + d5[7]))) + d5[8]
        t5[...] = jnp.maximum(z5 + c5[...], 0.0).astype(bf16)

    def stage_conv6_head(g):
        t5, tf = S[g][8], S[g][9]
        d6 = [jnp.dot(t5[pl.ds(kh * _G2 + kw, _R6), :], w6[3 * kh + kw],
                      preferred_element_type=f32)
              for kh in range(3) for kw in range(3)]
        z6 = (((d6[0] + d6[1]) + (d6[2] + d6[3]))
              + ((d6[4] + d6[5]) + (d6[6] + d6[7]))) + d6[8]
        r6 = jnp.maximum(z6 + c6[...], 0.0)
        ssum = None
        for i in range(_HEXT):
            band = jnp.sum(r6[_G2 * i:_G2 * i + _HEXT, :],
                           axis=0, keepdims=True)
            ssum = band if ssum is None else ssum + band
        pooled = ssum * (1.0 / (_HEXT * _HEXT))
        for b in range(_NB):
            tf[pl.ds(b, 1), :] = pooled[:, 16 * b:16 * (b + 1)]
        logits = jnp.dot(tf[...], wh[...],
                         preferred_element_type=f32) + bh[...]
        zc = logits - jnp.max(logits, axis=-1, keepdims=True)
        o_ref[pl.ds(g * _NB, _NB), :] = (
            zc - jnp.log(jnp.sum(jnp.exp(zc), axis=-1, keepdims=True)))

    for stage in (stage_x3, stage_conv1, stage_conv23, stage_pool,
                  stage_conv4, stage_conv5, stage_conv6_head):
        for g in range(_GRP):
            stage(g)


def _fold(gamma, beta, mean, var):
    scale = gamma / jnp.sqrt(var + _EPS)
    return scale, beta - mean * scale


def _block_taps(w, scale_in=None):
    w = w.astype(jnp.float32)
    if scale_in is not None:
        w = w * scale_in.astype(jnp.float32)[None, :, None, None]
    cout, cin = w.shape[0], w.shape[1]
    taps = jnp.transpose(w, (2, 3, 1, 0)).reshape(-1, cin, cout)
    eye = jnp.eye(_NB, dtype=jnp.float32)
    wbd = jnp.einsum("ab,tio->taibo", eye, taps)
    return wbd.reshape(taps.shape[0], _NB * cin, _NB * cout)


def _pair_w(taps):
    k, n = taps.shape[1], taps.shape[2]
    z = jnp.zeros((k, n), jnp.float32)
    cat = lambda a, b: jnp.concatenate([a, b], axis=0)
    return jnp.stack([cat(taps[0], taps[1]), cat(taps[3], taps[4]),
                      cat(taps[6], taps[7]), cat(taps[2], taps[5]),
                      cat(z, taps[8])])


def _shift_bias(w, shift_in):
    c = jnp.einsum("oikl,i->o", w.astype(jnp.float32),
                   shift_in.astype(jnp.float32))
    return jnp.tile(c, _NB).reshape(1, -1)


def kernel(x, w1_1, w1_2, w1_3, w2_1, w2_2, w2_3, w3,
           bn1_1_gamma, bn1_1_beta, bn1_1_mean, bn1_1_var,
           bn1_2_gamma, bn1_2_beta, bn1_2_mean, bn1_2_var,
           bn1_3_gamma, bn1_3_beta, bn1_3_mean, bn1_3_var,
           bn2_1_gamma, bn2_1_beta, bn2_1_mean, bn2_1_var,
           bn2_2_gamma, bn2_2_beta, bn2_2_mean, bn2_2_var,
           bn2_3_gamma, bn2_3_beta, bn2_3_mean, bn2_3_var):
    s1, sh1 = _fold(bn1_1_gamma, bn1_1_beta, bn1_1_mean, bn1_1_var)
    s2, sh2 = _fold(bn1_2_gamma, bn1_2_beta, bn1_2_mean, bn1_2_var)
    s3, sh3 = _fold(bn1_3_gamma, bn1_3_beta, bn1_3_mean, bn1_3_var)
    s4, sh4 = _fold(bn2_1_gamma, bn2_1_beta, bn2_1_mean, bn2_1_var)
    s5, sh5 = _fold(bn2_2_gamma, bn2_2_beta, bn2_2_mean, bn2_2_var)
    s6, sh6 = _fold(bn2_3_gamma, bn2_3_beta, bn2_3_mean, bn2_3_var)

    n = x.shape[0]
    per = _NB * _GRP
    steps = -(-n // per)
    n_pad = steps * per
    xf = x.astype(jnp.float32).reshape(n, _S1)
    if n_pad != n:
        xf = jnp.concatenate(
            [xf, jnp.zeros((n_pad - n, _S1), jnp.float32)], axis=0)
    xs = xf.reshape(steps, per, _S1)

    bf16 = jnp.bfloat16
    t1aps = _block_taps(w1_1)
    kpad = jnp.zeros((_L - 48, _L), jnp.float32)
    w1k = jnp.stack([jnp.concatenate([t1aps[3 * kh], t1aps[3 * kh + 1],
                                      t1aps[3 * kh + 2], kpad], axis=0)
                     for kh in range(3)]).astype(bf16)
    w2p = _pair_w(_block_taps(w1_2, scale_in=s1)).astype(bf16)
    c2 = _shift_bias(w1_2, sh1)
    w3b = _block_taps(w1_3, scale_in=s2).astype(bf16)
    c3 = _shift_bias(w1_3, sh2)
    w21p = jnp.concatenate(
        [w2_1.astype(jnp.float32),
         jnp.zeros((4,) + w2_1.shape[1:], jnp.float32)], axis=0)
    w4p = _pair_w(_block_taps(w21p)).astype(bf16)
    w22p = jnp.concatenate(
        [w2_2.astype(jnp.float32),
         jnp.zeros((w2_2.shape[0], 4) + w2_2.shape[2:], jnp.float32)], axis=1)
    s4p = jnp.concatenate([s4, jnp.ones((4,), jnp.float32)])
    w5b = _block_taps(w22p, scale_in=s4p).astype(bf16)
    c5 = _shift_bias(w2_2, sh4)
    w6b = _block_taps(w2_3, scale_in=s5).astype(bf16)
    c6 = _shift_bias(w2_3, sh5)
    s3t = jnp.tile(s3, _NB).reshape(1, -1)
    b3t = jnp.tile(sh3, _NB).reshape(1, -1)
    w3f = jnp.transpose(w3[:, :, 0, 0]).astype(jnp.float32)
    wh = w3f * s6.astype(jnp.float32)[:, None]
    bh = (sh6.astype(jnp.float32) @ w3f).reshape(1, 10)

    full = lambda *shape: pl.BlockSpec(shape, lambda s: (0,) * len(shape))
    in_specs = [
        pl.BlockSpec((None, _NB * _GRP, _S1), lambda s: (s, 0, 0)),
        full(3, _L, _L),
        full(5, 256, 256),
        full(1, 256),
        full(1, 256, _L),
        full(1, _L),
        full(1, _L),
        full(1, _L),
        full(5, 256, 256),
        full(9, 256, 256),
        full(1, 256),
        full(9, 256, 256),
        full(1, 256),
        full(16, 10),
        full(1, 10),
    ]
    args = (xs, w1k, w2p, c2, w3b, c3, s3t, b3t,
            w4p, w5b, c5, w6b, c6, wh, bh)

    group_scr = [
        pltpu.VMEM((_S1, _L), jnp.bfloat16),
        pltpu.VMEM((_R1, 2 * _L), jnp.bfloat16),
        pltpu.VMEM((_R1, 2 * _L), jnp.bfloat16),
        pltpu.VMEM((_R2, _L), jnp.float32),
        pltpu.VMEM((_R2, _L), jnp.float32),
        pltpu.VMEM((_RP, 2 * _L), jnp.bfloat16),
        pltpu.VMEM((_RP, 2 * _L), jnp.bfloat16),
        pltpu.VMEM((_R4, _NB * 16), jnp.bfloat16),
        pltpu.VMEM((_R5, _NB * 16), jnp.bfloat16),
        pltpu.VMEM((_NB, 16), jnp.float32),
    ]
    out = pl.pallas_call(
        _net_kernel,
        out_shape=jax.ShapeDtypeStruct((steps, _NB * _GRP, 10), jnp.float32),
        grid=(steps,),
        in_specs=in_specs,
        out_specs=pl.BlockSpec((None, _NB * _GRP, 10), lambda s: (s, 0, 0)),
        scratch_shapes=group_scr * _GRP,
        compiler_params=pltpu.CompilerParams(
            dimension_semantics=("parallel",),
            vmem_limit_bytes=48 * 1024 * 1024,
        ),
    )(*args)
    return out.reshape(n_pad, 10)[:n]


# speedup vs baseline: 3.3883x; 1.1308x over previous
import jax
import jax.numpy as jnp
from jax.experimental import pallas as pl
from jax.experimental.pallas import tpu as pltpu

_EPS = 1e-5

_G1 = 28
_G2 = 16
_S1 = _G1 * _G1

_R1 = 728
_R2 = 672
_RP = 192
_R4 = 154
_R5 = 120
_R6 = 86
_BANDS = 12
_HEXT = 6

_NB = 16
_GRP = 4
_L = _NB * 8


def _net_kernel(x_ref,
                w1k, w2p, c2, w3c, c3, s3, b3,
                w4p, w5, c5, w6, c6, wh, bh,
                o_ref, *scr):
    f32 = jnp.float32
    bf16 = jnp.bfloat16
    S = [scr[10 * g:10 * (g + 1)] for g in range(_GRP)]

    def stage_x3(g):
        x3 = S[g][0]
        xv = jnp.transpose(x_ref[pl.ds(g * _NB, _NB), :]).astype(bf16)
        z1r = jnp.zeros((1, _NB), bf16)
        x3[...] = jnp.concatenate(
            [xv,
             jnp.concatenate([xv[1:], z1r], axis=0),
             jnp.concatenate([xv[2:], z1r, z1r], axis=0),
             jnp.zeros((_S1, _L - 48), bf16)], axis=1)

    def stage_conv1(g):
        x3, ca, cb = S[g][0], S[g][1], S[g][2]
        z1 = (jnp.dot(x3[pl.ds(0, _R1), :], w1k[0],
                      preferred_element_type=f32)
              + jnp.dot(x3[pl.ds(_G1, _R1), :], w1k[1],
                        preferred_element_type=f32)
              + jnp.dot(x3[pl.ds(2 * _G1, _R1), :], w1k[2],
                        preferred_element_type=f32))
        v1h = jnp.maximum(z1, 0.0).astype(bf16)
        ca[...] = jnp.concatenate(
            [v1h, jnp.concatenate([v1h[1:], v1h[:1]], axis=0)], axis=1)
        cb[...] = jnp.concatenate(
            [v1h, jnp.concatenate([v1h[_G1:], v1h[:_G1]], axis=0)], axis=1)

    def stage_conv23(g):
        ca, cb, t3 = S[g][1], S[g][2], S[g][3]
        d0 = jnp.dot(ca[pl.ds(0, _R2), :], w2p[0], preferred_element_type=f32)
        d1 = jnp.dot(ca[pl.ds(_G1, _R2), :], w2p[1],
                     preferred_element_type=f32)
        d2 = jnp.dot(ca[pl.ds(2 * _G1, _R2), :], w2p[2],
                     preferred_element_type=f32)
        d3 = jnp.dot(cb[pl.ds(2, _R2), :], w2p[3], preferred_element_type=f32)
        d4 = jnp.dot(cb[pl.ds(_G1 + 2, _R2), :], w2p[4],
                     preferred_element_type=f32)
        z2 = ((d0 + d1) + (d2 + d3)) + d4
        y2 = jnp.maximum(z2 + c2[...], 0.0).astype(bf16)
        z3 = jnp.dot(y2, w3c[0], preferred_element_type=f32)
        t3[...] = jnp.maximum(z3 + c3[...], 0.0) * s3[...] + b3[...]

    def stage_pool(g):
        t3, ts, pa, pb = S[g][3], S[g][4], S[g][5], S[g][6]
        ts[pl.ds(0, _R2 - 1), :] = jnp.maximum(t3[pl.ds(0, _R2 - 1), :],
                                               t3[pl.ds(1, _R2 - 1), :])
        prev = None
        for i in range(_BANDS):
            base = 2 * i * _G1
            p = jnp.maximum(ts[pl.ds(base, _BANDS, 2), :],
                            ts[pl.ds(base + _G1, _BANDS, 2), :]).astype(bf16)
            pa[pl.ds(_G2 * i, _BANDS), :] = jnp.concatenate(
                [p, jnp.concatenate([p[1:], p[:1]], axis=0)], axis=1)
            if prev is not None:
                pb[pl.ds(_G2 * (i - 1), _BANDS), :] = jnp.concatenate(
                    [prev, p], axis=1)
            prev = p
        pb[pl.ds(_G2 * (_BANDS - 1), _BANDS), :] = jnp.concatenate(
            [prev, prev], axis=1)

    def stage_conv4(g):
        pa, pb, t4 = S[g][5], S[g][6], S[g][7]
        e0 = jnp.dot(pa[pl.ds(0, _R4), :], w4p[0], preferred_element_type=f32)
        e1 = jnp.dot(pa[pl.ds(_G2, _R4), :], w4p[1],
                     preferred_element_type=f32)
        e2 = jnp.dot(pa[pl.ds(2 * _G2, _R4), :], w4p[2],
                     preferred_element_type=f32)
        e3 = jnp.dot(pb[pl.ds(2, _R4), :], w4p[3], preferred_element_type=f32)
        e4 = jnp.dot(pb[pl.ds(_G2 + 2, _R4), :], w4p[4],
                     preferred_element_type=f32)
        z4 = ((e0 + e1) + (e2 + e3)) + e4
        t4[...] = jnp.maximum(z4, 0.0).astype(bf16)

    def stage_conv5(g):
        t4, t5 = S[g][7], S[g][8]
        d5 = [jnp.dot(t4[pl.ds(kh * _G2 + kw, _R5), :], w5[3 * kh + kw],
                      preferred_element_type=f32)
              for kh in range(3) for kw in range(3)]
        z5 = (((d5[0] + d5[1]) + (d5[2] + d5[3]))
              + ((d5[4] + d5[5]) + (d5[6] + d5[7]))) + d5[8]
        t5[...] = jnp.maximum(z5 + c5[...], 0.0).astype(bf16)

    def stage_conv6_head(g):
        t5, tf = S[g][8], S[g][9]
        d6 = [jnp.dot(t5[pl.ds(kh * _G2 + kw, _R6), :], w6[3 * kh + kw],
                      preferred_element_type=f32)
              for kh in range(3) for kw in range(3)]
        z6 = (((d6[0] + d6[1]) + (d6[2] + d6[3]))
              + ((d6[4] + d6[5]) + (d6[6] + d6[7]))) + d6[8]
        r6 = jnp.maximum(z6 + c6[...], 0.0)
        ssum = None
        for i in range(_HEXT):
            band = jnp.sum(r6[_G2 * i:_G2 * i + _HEXT, :],
                           axis=0, keepdims=True)
            ssum = band if ssum is None else ssum + band
        pooled = ssum * (1.0 / (_HEXT * _HEXT))
        for b in range(_NB):
            tf[pl.ds(b, 1), :] = pooled[:, 16 * b:16 * (b + 1)]
        logits = jnp.dot(tf[...], wh[...],
                         preferred_element_type=f32) + bh[...]
        zc = logits - jnp.max(logits, axis=-1, keepdims=True)
        o_ref[pl.ds(g * _NB, _NB), :] = (
            zc - jnp.log(jnp.sum(jnp.exp(zc), axis=-1, keepdims=True)))

    for stage in (stage_x3, stage_conv1, stage_conv23, stage_pool,
                  stage_conv4, stage_conv5, stage_conv6_head):
        for g in range(_GRP):
            stage(g)


def _fold(gamma, beta, mean, var):
    scale = gamma / jnp.sqrt(var + _EPS)
    return scale, beta - mean * scale


def _block_taps(w, scale_in=None):
    w = w.astype(jnp.float32)
    if scale_in is not None:
        w = w * scale_in.astype(jnp.float32)[None, :, None, None]
    cout, cin = w.shape[0], w.shape[1]
    taps = jnp.transpose(w, (2, 3, 1, 0)).reshape(-1, cin, cout)
    eye = jnp.eye(_NB, dtype=jnp.float32)
    wbd = jnp.einsum("ab,tio->taibo", eye, taps)
    return wbd.reshape(taps.shape[0], _NB * cin, _NB * cout)


def _pair_w(taps):
    k, n = taps.shape[1], taps.shape[2]
    z = jnp.zeros((k, n), jnp.float32)
    cat = lambda a, b: jnp.concatenate([a, b], axis=0)
    return jnp.stack([cat(taps[0], taps[1]), cat(taps[3], taps[4]),
                      cat(taps[6], taps[7]), cat(taps[2], taps[5]),
                      cat(z, taps[8])])


def _shift_bias(w, shift_in):
    c = jnp.einsum("oikl,i->o", w.astype(jnp.float32),
                   shift_in.astype(jnp.float32))
    return jnp.tile(c, _NB).reshape(1, -1)


def kernel(x, w1_1, w1_2, w1_3, w2_1, w2_2, w2_3, w3,
           bn1_1_gamma, bn1_1_beta, bn1_1_mean, bn1_1_var,
           bn1_2_gamma, bn1_2_beta, bn1_2_mean, bn1_2_var,
           bn1_3_gamma, bn1_3_beta, bn1_3_mean, bn1_3_var,
           bn2_1_gamma, bn2_1_beta, bn2_1_mean, bn2_1_var,
           bn2_2_gamma, bn2_2_beta, bn2_2_mean, bn2_2_var,
           bn2_3_gamma, bn2_3_beta, bn2_3_mean, bn2_3_var):
    s1, sh1 = _fold(bn1_1_gamma, bn1_1_beta, bn1_1_mean, bn1_1_var)
    s2, sh2 = _fold(bn1_2_gamma, bn1_2_beta, bn1_2_mean, bn1_2_var)
    s3, sh3 = _fold(bn1_3_gamma, bn1_3_beta, bn1_3_mean, bn1_3_var)
    s4, sh4 = _fold(bn2_1_gamma, bn2_1_beta, bn2_1_mean, bn2_1_var)
    s5, sh5 = _fold(bn2_2_gamma, bn2_2_beta, bn2_2_mean, bn2_2_var)
    s6, sh6 = _fold(bn2_3_gamma, bn2_3_beta, bn2_3_mean, bn2_3_var)

    n = x.shape[0]
    per = _NB * _GRP
    steps = -(-n // per)
    n_pad = steps * per
    xf = x.astype(jnp.float32).reshape(n, _S1)
    if n_pad != n:
        xf = jnp.concatenate(
            [xf, jnp.zeros((n_pad - n, _S1), jnp.float32)], axis=0)
    xs = xf.reshape(steps, per, _S1)

    bf16 = jnp.bfloat16
    t1aps = _block_taps(w1_1)
    kpad = jnp.zeros((_L - 48, _L), jnp.float32)
    w1k = jnp.stack([jnp.concatenate([t1aps[3 * kh], t1aps[3 * kh + 1],
                                      t1aps[3 * kh + 2], kpad], axis=0)
                     for kh in range(3)]).astype(bf16)
    w2p = _pair_w(_block_taps(w1_2, scale_in=s1)).astype(bf16)
    c2 = _shift_bias(w1_2, sh1)
    w3b = _block_taps(w1_3, scale_in=s2).astype(bf16)
    c3 = _shift_bias(w1_3, sh2)
    w21p = jnp.concatenate(
        [w2_1.astype(jnp.float32),
         jnp.zeros((4,) + w2_1.shape[1:], jnp.float32)], axis=0)
    w4p = _pair_w(_block_taps(w21p)).astype(bf16)
    w22p = jnp.concatenate(
        [w2_2.astype(jnp.float32),
         jnp.zeros((w2_2.shape[0], 4) + w2_2.shape[2:], jnp.float32)], axis=1)
    s4p = jnp.concatenate([s4, jnp.ones((4,), jnp.float32)])
    w5b = _block_taps(w22p, scale_in=s4p).astype(bf16)
    c5 = _shift_bias(w2_2, sh4)
    w6b = _block_taps(w2_3, scale_in=s5).astype(bf16)
    c6 = _shift_bias(w2_3, sh5)
    s3t = jnp.tile(s3, _NB).reshape(1, -1)
    b3t = jnp.tile(sh3, _NB).reshape(1, -1)
    w3f = jnp.transpose(w3[:, :, 0, 0]).astype(jnp.float32)
    wh = w3f * s6.astype(jnp.float32)[:, None]
    bh = (sh6.astype(jnp.float32) @ w3f).reshape(1, 10)

    full = lambda *shape: pl.BlockSpec(shape, lambda s: (0,) * len(shape))
    in_specs = [
        pl.BlockSpec((None, _NB * _GRP, _S1), lambda s: (s, 0, 0)),
        full(3, _L, _L),
        full(5, 256, 256),
        full(1, 256),
        full(1, 256, _L),
        full(1, _L),
        full(1, _L),
        full(1, _L),
        full(5, 256, 256),
        full(9, 256, 256),
        full(1, 256),
        full(9, 256, 256),
        full(1, 256),
        full(16, 10),
        full(1, 10),
    ]
    args = (xs, w1k, w2p, c2, w3b, c3, s3t, b3t,
            w4p, w5b, c5, w6b, c6, wh, bh)

    group_scr = [
        pltpu.VMEM((_S1, _L), jnp.bfloat16),
        pltpu.VMEM((_R1, 2 * _L), jnp.bfloat16),
        pltpu.VMEM((_R1, 2 * _L), jnp.bfloat16),
        pltpu.VMEM((_R2, _L), jnp.float32),
        pltpu.VMEM((_R2, _L), jnp.float32),
        pltpu.VMEM((_RP, 2 * _L), jnp.bfloat16),
        pltpu.VMEM((_RP, 2 * _L), jnp.bfloat16),
        pltpu.VMEM((_R4, _NB * 16), jnp.bfloat16),
        pltpu.VMEM((_R5, _NB * 16), jnp.bfloat16),
        pltpu.VMEM((_NB, 16), jnp.float32),
    ]
    out = pl.pallas_call(
        _net_kernel,
        out_shape=jax.ShapeDtypeStruct((steps, _NB * _GRP, 10), jnp.float32),
        grid=(steps,),
        in_specs=in_specs,
        out_specs=pl.BlockSpec((None, _NB * _GRP, 10), lambda s: (s, 0, 0)),
        scratch_shapes=group_scr * _GRP,
        compiler_params=pltpu.CompilerParams(
            dimension_semantics=("parallel",),
            vmem_limit_bytes=48 * 1024 * 1024,
        ),
    )(*args)
    return out.reshape(n_pad, 10)[:n]


# speedup vs baseline: 3.5687x; 1.0532x over previous
import jax
import jax.numpy as jnp
from jax.experimental import pallas as pl
from jax.experimental.pallas import tpu as pltpu

_EPS = 1e-5

_G1 = 28
_G2 = 16
_S1 = _G1 * _G1

_R1 = 728
_R2 = 672
_RP = 192
_R4 = 154
_R5 = 120
_R6 = 86
_BANDS = 12
_HEXT = 6

_NB = 16
_GRP = 4
_L = _NB * 8


def _net_kernel(x_ref,
                w1k, w2p, c2, w3c, c3, s3, b3,
                w4p, w5, c5, w6, c6, wh, bh,
                o_ref, *scr):
    f32 = jnp.float32
    bf16 = jnp.bfloat16
    S = [scr[10 * g:10 * (g + 1)] for g in range(_GRP)]

    def stage_x3(g):
        x3 = S[g][0]
        xv = jnp.transpose(x_ref[pl.ds(g * _NB, _NB), :]).astype(bf16)
        z1r = jnp.zeros((1, _NB), bf16)
        x3[...] = jnp.concatenate(
            [xv,
             jnp.concatenate([xv[1:], z1r], axis=0),
             jnp.concatenate([xv[2:], z1r, z1r], axis=0),
             jnp.zeros((_S1, _L - 48), bf16)], axis=1)

    def stage_conv1(g):
        x3, ca, cb = S[g][0], S[g][1], S[g][2]
        z1 = (jnp.dot(x3[pl.ds(0, _R1), :], w1k[0],
                      preferred_element_type=f32)
              + jnp.dot(x3[pl.ds(_G1, _R1), :], w1k[1],
                        preferred_element_type=f32)
              + jnp.dot(x3[pl.ds(2 * _G1, _R1), :], w1k[2],
                        preferred_element_type=f32))
        v1h = jnp.maximum(z1, 0.0).astype(bf16)
        ca[...] = jnp.concatenate(
            [v1h, jnp.concatenate([v1h[1:], v1h[:1]], axis=0)], axis=1)
        cb[...] = jnp.concatenate(
            [v1h, jnp.concatenate([v1h[_G1:], v1h[:_G1]], axis=0)], axis=1)

    def stage_conv23(g):
        ca, cb, t3 = S[g][1], S[g][2], S[g][3]
        d0 = jnp.dot(ca[pl.ds(0, _R2), :], w2p[0], preferred_element_type=f32)
        d1 = jnp.dot(ca[pl.ds(_G1, _R2), :], w2p[1],
                     preferred_element_type=f32)
        d2 = jnp.dot(ca[pl.ds(2 * _G1, _R2), :], w2p[2],
                     preferred_element_type=f32)
        d3 = jnp.dot(cb[pl.ds(2, _R2), :], w2p[3], preferred_element_type=f32)
        d4 = jnp.dot(cb[pl.ds(_G1 + 2, _R2), :], w2p[4],
                     preferred_element_type=f32)
        z2 = ((d0 + d1) + (d2 + d3)) + d4
        y2 = jnp.maximum(z2 + c2[...], 0.0).astype(bf16)
        z3 = jnp.dot(y2, w3c[0], preferred_element_type=f32)
        t3[...] = jnp.maximum(z3 + c3[...], 0.0) * s3[...] + b3[...]

    def stage_pool(g):
        t3, ts, pa, pb = S[g][3], S[g][4], S[g][5], S[g][6]
        ts[pl.ds(0, _R2 - 1), :] = jnp.maximum(t3[pl.ds(0, _R2 - 1), :],
                                               t3[pl.ds(1, _R2 - 1), :])
        prev = None
        for i in range(_BANDS):
            base = 2 * i * _G1
            p = jnp.maximum(ts[pl.ds(base, _BANDS, 2), :],
                            ts[pl.ds(base + _G1, _BANDS, 2), :]).astype(bf16)
            pa[pl.ds(_G2 * i, _BANDS), :] = jnp.concatenate(
                [p, jnp.concatenate([p[1:], p[:1]], axis=0)], axis=1)
            if prev is not None:
                pb[pl.ds(_G2 * (i - 1), _BANDS), :] = jnp.concatenate(
                    [prev, p], axis=1)
            prev = p
        pb[pl.ds(_G2 * (_BANDS - 1), _BANDS), :] = jnp.concatenate(
            [prev, prev], axis=1)

    def _acc(zs, g, d):
        zs[g] = d if zs[g] is None else zs[g] + d

    def stage_conv4_all():
        zs = [None] * _GRP
        for t, (buf, off) in enumerate(
                ((5, 0), (5, _G2), (5, 2 * _G2), (6, 2), (6, _G2 + 2))):
            wt = w4p[t]
            for g in range(_GRP):
                _acc(zs, g, jnp.dot(S[g][buf][pl.ds(off, _R4), :], wt,
                                    preferred_element_type=f32))
        for g in range(_GRP):
            S[g][7][...] = jnp.maximum(zs[g], 0.0).astype(bf16)

    def stage_conv5_all():
        zs = [None] * _GRP
        for t in range(9):
            wt = w5[t]
            off = (t // 3) * _G2 + t % 3
            for g in range(_GRP):
                _acc(zs, g, jnp.dot(S[g][7][pl.ds(off, _R5), :], wt,
                                    preferred_element_type=f32))
        for g in range(_GRP):
            S[g][8][...] = jnp.maximum(zs[g] + c5[...], 0.0).astype(bf16)

    def stage_conv6_head_all():
        zs = [None] * _GRP
        for t in range(9):
            wt = w6[t]
            off = (t // 3) * _G2 + t % 3
            for g in range(_GRP):
                _acc(zs, g, jnp.dot(S[g][8][pl.ds(off, _R6), :], wt,
                                    preferred_element_type=f32))
        for g in range(_GRP):
            tf = S[g][9]
            r6 = jnp.maximum(zs[g] + c6[...], 0.0)
            ssum = None
            for i in range(_HEXT):
                band = jnp.sum(r6[_G2 * i:_G2 * i + _HEXT, :],
                               axis=0, keepdims=True)
                ssum = band if ssum is None else ssum + band
            pooled = ssum * (1.0 / (_HEXT * _HEXT))
            for b in range(_NB):
                tf[pl.ds(b, 1), :] = pooled[:, 16 * b:16 * (b + 1)]
            logits = jnp.dot(tf[...], wh[...],
                             preferred_element_type=f32) + bh[...]
            zc = logits - jnp.max(logits, axis=-1, keepdims=True)
            o_ref[pl.ds(g * _NB, _NB), :] = (
                zc - jnp.log(jnp.sum(jnp.exp(zc), axis=-1, keepdims=True)))

    for stage in (stage_x3, stage_conv1, stage_conv23, stage_pool):
        for g in range(_GRP):
            stage(g)
    stage_conv4_all()
    stage_conv5_all()
    stage_conv6_head_all()


def _fold(gamma, beta, mean, var):
    scale = gamma / jnp.sqrt(var + _EPS)
    return scale, beta - mean * scale


def _block_taps(w, scale_in=None):
    w = w.astype(jnp.float32)
    if scale_in is not None:
        w = w * scale_in.astype(jnp.float32)[None, :, None, None]
    cout, cin = w.shape[0], w.shape[1]
    taps = jnp.transpose(w, (2, 3, 1, 0)).reshape(-1, cin, cout)
    eye = jnp.eye(_NB, dtype=jnp.float32)
    wbd = jnp.einsum("ab,tio->taibo", eye, taps)
    return wbd.reshape(taps.shape[0], _NB * cin, _NB * cout)


def _pair_w(taps):
    k, n = taps.shape[1], taps.shape[2]
    z = jnp.zeros((k, n), jnp.float32)
    cat = lambda a, b: jnp.concatenate([a, b], axis=0)
    return jnp.stack([cat(taps[0], taps[1]), cat(taps[3], taps[4]),
                      cat(taps[6], taps[7]), cat(taps[2], taps[5]),
                      cat(z, taps[8])])


def _shift_bias(w, shift_in):
    c = jnp.einsum("oikl,i->o", w.astype(jnp.float32),
                   shift_in.astype(jnp.float32))
    return jnp.tile(c, _NB).reshape(1, -1)


def kernel(x, w1_1, w1_2, w1_3, w2_1, w2_2, w2_3, w3,
           bn1_1_gamma, bn1_1_beta, bn1_1_mean, bn1_1_var,
           bn1_2_gamma, bn1_2_beta, bn1_2_mean, bn1_2_var,
           bn1_3_gamma, bn1_3_beta, bn1_3_mean, bn1_3_var,
           bn2_1_gamma, bn2_1_beta, bn2_1_mean, bn2_1_var,
           bn2_2_gamma, bn2_2_beta, bn2_2_mean, bn2_2_var,
           bn2_3_gamma, bn2_3_beta, bn2_3_mean, bn2_3_var):
    s1, sh1 = _fold(bn1_1_gamma, bn1_1_beta, bn1_1_mean, bn1_1_var)
    s2, sh2 = _fold(bn1_2_gamma, bn1_2_beta, bn1_2_mean, bn1_2_var)
    s3, sh3 = _fold(bn1_3_gamma, bn1_3_beta, bn1_3_mean, bn1_3_var)
    s4, sh4 = _fold(bn2_1_gamma, bn2_1_beta, bn2_1_mean, bn2_1_var)
    s5, sh5 = _fold(bn2_2_gamma, bn2_2_beta, bn2_2_mean, bn2_2_var)
    s6, sh6 = _fold(bn2_3_gamma, bn2_3_beta, bn2_3_mean, bn2_3_var)

    n = x.shape[0]
    per = _NB * _GRP
    steps = -(-n // per)
    n_pad = steps * per
    xf = x.astype(jnp.float32).reshape(n, _S1)
    if n_pad != n:
        xf = jnp.concatenate(
            [xf, jnp.zeros((n_pad - n, _S1), jnp.float32)], axis=0)
    xs = xf.reshape(steps, per, _S1)

    bf16 = jnp.bfloat16
    t1aps = _block_taps(w1_1)
    kpad = jnp.zeros((_L - 48, _L), jnp.float32)
    w1k = jnp.stack([jnp.concatenate([t1aps[3 * kh], t1aps[3 * kh + 1],
                                      t1aps[3 * kh + 2], kpad], axis=0)
                     for kh in range(3)]).astype(bf16)
    w2p = _pair_w(_block_taps(w1_2, scale_in=s1)).astype(bf16)
    c2 = _shift_bias(w1_2, sh1)
    w3b = _block_taps(w1_3, scale_in=s2).astype(bf16)
    c3 = _shift_bias(w1_3, sh2)
    w21p = jnp.concatenate(
        [w2_1.astype(jnp.float32),
         jnp.zeros((4,) + w2_1.shape[1:], jnp.float32)], axis=0)
    w4p = _pair_w(_block_taps(w21p)).astype(bf16)
    w22p = jnp.concatenate(
        [w2_2.astype(jnp.float32),
         jnp.zeros((w2_2.shape[0], 4) + w2_2.shape[2:], jnp.float32)], axis=1)
    s4p = jnp.concatenate([s4, jnp.ones((4,), jnp.float32)])
    w5b = _block_taps(w22p, scale_in=s4p).astype(bf16)
    c5 = _shift_bias(w2_2, sh4)
    w6b = _block_taps(w2_3, scale_in=s5).astype(bf16)
    c6 = _shift_bias(w2_3, sh5)
    s3t = jnp.tile(s3, _NB).reshape(1, -1)
    b3t = jnp.tile(sh3, _NB).reshape(1, -1)
    w3f = jnp.transpose(w3[:, :, 0, 0]).astype(jnp.float32)
    wh = w3f * s6.astype(jnp.float32)[:, None]
    bh = (sh6.astype(jnp.float32) @ w3f).reshape(1, 10)

    full = lambda *shape: pl.BlockSpec(shape, lambda s: (0,) * len(shape))
    in_specs = [
        pl.BlockSpec((None, _NB * _GRP, _S1), lambda s: (s, 0, 0)),
        full(3, _L, _L),
        full(5, 256, 256),
        full(1, 256),
        full(1, 256, _L),
        full(1, _L),
        full(1, _L),
        full(1, _L),
        full(5, 256, 256),
        full(9, 256, 256),
        full(1, 256),
        full(9, 256, 256),
        full(1, 256),
        full(16, 10),
        full(1, 10),
    ]
    args = (xs, w1k, w2p, c2, w3b, c3, s3t, b3t,
            w4p, w5b, c5, w6b, c6, wh, bh)

    group_scr = [
        pltpu.VMEM((_S1, _L), jnp.bfloat16),
        pltpu.VMEM((_R1, 2 * _L), jnp.bfloat16),
        pltpu.VMEM((_R1, 2 * _L), jnp.bfloat16),
        pltpu.VMEM((_R2, _L), jnp.float32),
        pltpu.VMEM((_R2, _L), jnp.float32),
        pltpu.VMEM((_RP, 2 * _L), jnp.bfloat16),
        pltpu.VMEM((_RP, 2 * _L), jnp.bfloat16),
        pltpu.VMEM((_R4, _NB * 16), jnp.bfloat16),
        pltpu.VMEM((_R5, _NB * 16), jnp.bfloat16),
        pltpu.VMEM((_NB, 16), jnp.float32),
    ]
    out = pl.pallas_call(
        _net_kernel,
        out_shape=jax.ShapeDtypeStruct((steps, _NB * _GRP, 10), jnp.float32),
        grid=(steps,),
        in_specs=in_specs,
        out_specs=pl.BlockSpec((None, _NB * _GRP, 10), lambda s: (s, 0, 0)),
        scratch_shapes=group_scr * _GRP,
        compiler_params=pltpu.CompilerParams(
            dimension_semantics=("parallel",),
            vmem_limit_bytes=48 * 1024 * 1024,
        ),
    )(*args)
    return out.reshape(n_pad, 10)[:n]


# speedup vs baseline: 3.6789x; 1.0309x over previous
import jax
import jax.numpy as jnp
from jax.experimental import pallas as pl
from jax.experimental.pallas import tpu as pltpu

_EPS = 1e-5

_G1 = 28
_G2 = 16
_S1 = _G1 * _G1

_R1 = 728
_R2 = 672
_RP = 192
_R4 = 154
_R5 = 120
_R6 = 86
_BANDS = 12
_HEXT = 6

_NB = 16
_GRP = 4
_L = _NB * 8


def _net_kernel(x_ref,
                w1k, w2p, c2, w3c, c3, s3, b3,
                w4p, w5, c5, w6, c6, wh, bh,
                o_ref, *scr):
    f32 = jnp.float32
    bf16 = jnp.bfloat16
    S = [scr[10 * g:10 * (g + 1)] for g in range(_GRP)]

    def stage_x3(g):
        x3 = S[g][0]
        xv = jnp.transpose(x_ref[pl.ds(g * _NB, _NB), :])
        z1r = jnp.zeros((1, _NB), bf16)
        x3[...] = jnp.concatenate(
            [xv,
             jnp.concatenate([xv[1:], z1r], axis=0),
             jnp.concatenate([xv[2:], z1r, z1r], axis=0),
             jnp.zeros((_S1, _L - 48), bf16)], axis=1)

    def stage_conv1(g):
        x3, ca, cb = S[g][0], S[g][1], S[g][2]
        z1 = (jnp.dot(x3[pl.ds(0, _R1), :], w1k[0],
                      preferred_element_type=f32)
              + jnp.dot(x3[pl.ds(_G1, _R1), :], w1k[1],
                        preferred_element_type=f32)
              + jnp.dot(x3[pl.ds(2 * _G1, _R1), :], w1k[2],
                        preferred_element_type=f32))
        v1h = jnp.maximum(z1, 0.0).astype(bf16)
        ca[...] = jnp.concatenate(
            [v1h, jnp.concatenate([v1h[1:], v1h[:1]], axis=0)], axis=1)
        cb[...] = jnp.concatenate(
            [v1h, jnp.concatenate([v1h[_G1:], v1h[:_G1]], axis=0)], axis=1)

    def stage_conv23(g):
        ca, cb, t3 = S[g][1], S[g][2], S[g][3]
        d0 = jnp.dot(ca[pl.ds(0, _R2), :], w2p[0], preferred_element_type=f32)
        d1 = jnp.dot(ca[pl.ds(_G1, _R2), :], w2p[1],
                     preferred_element_type=f32)
        d2 = jnp.dot(ca[pl.ds(2 * _G1, _R2), :], w2p[2],
                     preferred_element_type=f32)
        d3 = jnp.dot(cb[pl.ds(2, _R2), :], w2p[3], preferred_element_type=f32)
        d4 = jnp.dot(cb[pl.ds(_G1 + 2, _R2), :], w2p[4],
                     preferred_element_type=f32)
        z2 = ((d0 + d1) + (d2 + d3)) + d4
        y2 = jnp.maximum(z2 + c2[...], 0.0).astype(bf16)
        z3 = jnp.dot(y2, w3c[0], preferred_element_type=f32)
        t3[...] = jnp.maximum(z3 + c3[...], 0.0) * s3[...] + b3[...]

    def stage_pool(g):
        t3, ts, pa, pb = S[g][3], S[g][4], S[g][5], S[g][6]
        ts[pl.ds(0, _R2 - 1), :] = jnp.maximum(t3[pl.ds(0, _R2 - 1), :],
                                               t3[pl.ds(1, _R2 - 1), :])
        prev = None
        for i in range(_BANDS):
            base = 2 * i * _G1
            p = jnp.maximum(ts[pl.ds(base, _BANDS, 2), :],
                            ts[pl.ds(base + _G1, _BANDS, 2), :]).astype(bf16)
            pa[pl.ds(_G2 * i, _BANDS), :] = jnp.concatenate(
                [p, jnp.concatenate([p[1:], p[:1]], axis=0)], axis=1)
            if prev is not None:
                pb[pl.ds(_G2 * (i - 1), _BANDS), :] = jnp.concatenate(
                    [prev, p], axis=1)
            prev = p
        pb[pl.ds(_G2 * (_BANDS - 1), _BANDS), :] = jnp.concatenate(
            [prev, prev], axis=1)

    def _acc(zs, g, d):
        zs[g] = d if zs[g] is None else zs[g] + d

    def stage_conv4_all():
        zs = [None] * _GRP
        for t, (buf, off) in enumerate(
                ((5, 0), (5, _G2), (5, 2 * _G2), (6, 2), (6, _G2 + 2))):
            wt = w4p[t]
            for g in range(_GRP):
                _acc(zs, g, jnp.dot(S[g][buf][pl.ds(off, _R4), :], wt,
                                    preferred_element_type=f32))
        for g in range(_GRP):
            S[g][7][...] = jnp.maximum(zs[g], 0.0).astype(bf16)

    def stage_conv5_all():
        zs = [None] * _GRP
        for t in range(9):
            wt = w5[t]
            off = (t // 3) * _G2 + t % 3
            for g in range(_GRP):
                _acc(zs, g, jnp.dot(S[g][7][pl.ds(off, _R5), :], wt,
                                    preferred_element_type=f32))
        for g in range(_GRP):
            S[g][8][...] = jnp.maximum(zs[g] + c5[...], 0.0).astype(bf16)

    def stage_conv6_head_all():
        zs = [None] * _GRP
        for t in range(9):
            wt = w6[t]
            off = (t // 3) * _G2 + t % 3
            for g in range(_GRP):
                _acc(zs, g, jnp.dot(S[g][8][pl.ds(off, _R6), :], wt,
                                    preferred_element_type=f32))
        for g in range(_GRP):
            tf = S[g][9]
            r6 = jnp.maximum(zs[g] + c6[...], 0.0)
            ssum = None
            for i in range(_HEXT):
                band = jnp.sum(r6[_G2 * i:_G2 * i + _HEXT, :],
                               axis=0, keepdims=True)
                ssum = band if ssum is None else ssum + band
            pooled = ssum * (1.0 / (_HEXT * _HEXT))
            for b in range(_NB):
                tf[pl.ds(b, 1), :] = pooled[:, 16 * b:16 * (b + 1)]
            logits = jnp.dot(tf[...], wh[...],
                             preferred_element_type=f32) + bh[...]
            zc = logits - jnp.max(logits, axis=-1, keepdims=True)
            o_ref[pl.ds(g * _NB, _NB), :] = (
                zc - jnp.log(jnp.sum(jnp.exp(zc), axis=-1, keepdims=True)))

    for stage in (stage_x3, stage_conv1, stage_conv23, stage_pool):
        for g in range(_GRP):
            stage(g)
    stage_conv4_all()
    stage_conv5_all()
    stage_conv6_head_all()


def _fold(gamma, beta, mean, var):
    scale = gamma / jnp.sqrt(var + _EPS)
    return scale, beta - mean * scale


def _block_taps(w, scale_in=None):
    w = w.astype(jnp.float32)
    if scale_in is not None:
        w = w * scale_in.astype(jnp.float32)[None, :, None, None]
    cout, cin = w.shape[0], w.shape[1]
    taps = jnp.transpose(w, (2, 3, 1, 0)).reshape(-1, cin, cout)
    eye = jnp.eye(_NB, dtype=jnp.float32)
    wbd = jnp.einsum("ab,tio->taibo", eye, taps)
    return wbd.reshape(taps.shape[0], _NB * cin, _NB * cout)


def _pair_w(taps):
    k, n = taps.shape[1], taps.shape[2]
    z = jnp.zeros((k, n), jnp.float32)
    cat = lambda a, b: jnp.concatenate([a, b], axis=0)
    return jnp.stack([cat(taps[0], taps[1]), cat(taps[3], taps[4]),
                      cat(taps[6], taps[7]), cat(taps[2], taps[5]),
                      cat(z, taps[8])])


def _shift_bias(w, shift_in):
    c = jnp.einsum("oikl,i->o", w.astype(jnp.float32),
                   shift_in.astype(jnp.float32))
    return jnp.tile(c, _NB).reshape(1, -1)


def kernel(x, w1_1, w1_2, w1_3, w2_1, w2_2, w2_3, w3,
           bn1_1_gamma, bn1_1_beta, bn1_1_mean, bn1_1_var,
           bn1_2_gamma, bn1_2_beta, bn1_2_mean, bn1_2_var,
           bn1_3_gamma, bn1_3_beta, bn1_3_mean, bn1_3_var,
           bn2_1_gamma, bn2_1_beta, bn2_1_mean, bn2_1_var,
           bn2_2_gamma, bn2_2_beta, bn2_2_mean, bn2_2_var,
           bn2_3_gamma, bn2_3_beta, bn2_3_mean, bn2_3_var):
    s1, sh1 = _fold(bn1_1_gamma, bn1_1_beta, bn1_1_mean, bn1_1_var)
    s2, sh2 = _fold(bn1_2_gamma, bn1_2_beta, bn1_2_mean, bn1_2_var)
    s3, sh3 = _fold(bn1_3_gamma, bn1_3_beta, bn1_3_mean, bn1_3_var)
    s4, sh4 = _fold(bn2_1_gamma, bn2_1_beta, bn2_1_mean, bn2_1_var)
    s5, sh5 = _fold(bn2_2_gamma, bn2_2_beta, bn2_2_mean, bn2_2_var)
    s6, sh6 = _fold(bn2_3_gamma, bn2_3_beta, bn2_3_mean, bn2_3_var)

    n = x.shape[0]
    per = _NB * _GRP
    steps = -(-n // per)
    n_pad = steps * per
    xf = x.astype(jnp.float32).reshape(n, _S1)
    if n_pad != n:
        xf = jnp.concatenate(
            [xf, jnp.zeros((n_pad - n, _S1), jnp.float32)], axis=0)
    xs = xf.reshape(steps, per, _S1).astype(jnp.bfloat16)

    bf16 = jnp.bfloat16
    t1aps = _block_taps(w1_1)
    kpad = jnp.zeros((_L - 48, _L), jnp.float32)
    w1k = jnp.stack([jnp.concatenate([t1aps[3 * kh], t1aps[3 * kh + 1],
                                      t1aps[3 * kh + 2], kpad], axis=0)
                     for kh in range(3)]).astype(bf16)
    w2p = _pair_w(_block_taps(w1_2, scale_in=s1)).astype(bf16)
    c2 = _shift_bias(w1_2, sh1)
    w3b = _block_taps(w1_3, scale_in=s2).astype(bf16)
    c3 = _shift_bias(w1_3, sh2)
    w21p = jnp.concatenate(
        [w2_1.astype(jnp.float32),
         jnp.zeros((4,) + w2_1.shape[1:], jnp.float32)], axis=0)
    w4p = _pair_w(_block_taps(w21p)).astype(bf16)
    w22p = jnp.concatenate(
        [w2_2.astype(jnp.float32),
         jnp.zeros((w2_2.shape[0], 4) + w2_2.shape[2:], jnp.float32)], axis=1)
    s4p = jnp.concatenate([s4, jnp.ones((4,), jnp.float32)])
    w5b = _block_taps(w22p, scale_in=s4p).astype(bf16)
    c5 = _shift_bias(w2_2, sh4)
    w6b = _block_taps(w2_3, scale_in=s5).astype(bf16)
    c6 = _shift_bias(w2_3, sh5)
    s3t = jnp.tile(s3, _NB).reshape(1, -1)
    b3t = jnp.tile(sh3, _NB).reshape(1, -1)
    w3f = jnp.transpose(w3[:, :, 0, 0]).astype(jnp.float32)
    wh = w3f * s6.astype(jnp.float32)[:, None]
    bh = (sh6.astype(jnp.float32) @ w3f).reshape(1, 10)

    full = lambda *shape: pl.BlockSpec(shape, lambda s: (0,) * len(shape))
    in_specs = [
        pl.BlockSpec((None, _NB * _GRP, _S1), lambda s: (s, 0, 0)),
        full(3, _L, _L),
        full(5, 256, 256),
        full(1, 256),
        full(1, 256, _L),
        full(1, _L),
        full(1, _L),
        full(1, _L),
        full(5, 256, 256),
        full(9, 256, 256),
        full(1, 256),
        full(9, 256, 256),
        full(1, 256),
        full(16, 10),
        full(1, 10),
    ]
    args = (xs, w1k, w2p, c2, w3b, c3, s3t, b3t,
            w4p, w5b, c5, w6b, c6, wh, bh)

    group_scr = [
        pltpu.VMEM((_S1, _L), jnp.bfloat16),
        pltpu.VMEM((_R1, 2 * _L), jnp.bfloat16),
        pltpu.VMEM((_R1, 2 * _L), jnp.bfloat16),
        pltpu.VMEM((_R2, _L), jnp.float32),
        pltpu.VMEM((_R2, _L), jnp.float32),
        pltpu.VMEM((_RP, 2 * _L), jnp.bfloat16),
        pltpu.VMEM((_RP, 2 * _L), jnp.bfloat16),
        pltpu.VMEM((_R4, _NB * 16), jnp.bfloat16),
        pltpu.VMEM((_R5, _NB * 16), jnp.bfloat16),
        pltpu.VMEM((_NB, 16), jnp.float32),
    ]
    out = pl.pallas_call(
        _net_kernel,
        out_shape=jax.ShapeDtypeStruct((steps, _NB * _GRP, 10), jnp.float32),
        grid=(steps,),
        in_specs=in_specs,
        out_specs=pl.BlockSpec((None, _NB * _GRP, 10), lambda s: (s, 0, 0)),
        scratch_shapes=group_scr * _GRP,
        compiler_params=pltpu.CompilerParams(
            dimension_semantics=("parallel",),
            vmem_limit_bytes=48 * 1024 * 1024,
        ),
    )(*args)
    return out.reshape(n_pad, 10)[:n]


# speedup vs baseline: 3.8963x; 1.0591x over previous
import jax
import jax.numpy as jnp
from jax.experimental import pallas as pl
from jax.experimental.pallas import tpu as pltpu

_EPS = 1e-5

_G1 = 28
_G2 = 16
_S1 = _G1 * _G1

_R1 = 728
_R2 = 672
_RP = 192
_R4 = 154
_R5 = 120
_R6 = 86
_BANDS = 12
_HEXT = 6

_NB = 16
_GRP = 6
_L = _NB * 8


def _net_kernel(x_ref,
                w1k, w2p, c2, w3c, c3, s3, b3,
                w4p, w5, c5, w6, c6, wh, bh,
                o_ref, *scr):
    f32 = jnp.float32
    bf16 = jnp.bfloat16
    S = [scr[10 * g:10 * (g + 1)] for g in range(_GRP)]

    def stage_x3(g):
        x3 = S[g][0]
        xv = jnp.transpose(x_ref[pl.ds(g * _NB, _NB), :])
        z1r = jnp.zeros((1, _NB), bf16)
        x3[...] = jnp.concatenate(
            [xv,
             jnp.concatenate([xv[1:], z1r], axis=0),
             jnp.concatenate([xv[2:], z1r, z1r], axis=0),
             jnp.zeros((_S1, _L - 48), bf16)], axis=1)

    def stage_conv1(g):
        x3, ca, cb = S[g][0], S[g][1], S[g][2]
        z1 = (jnp.dot(x3[pl.ds(0, _R1), :], w1k[0],
                      preferred_element_type=f32)
              + jnp.dot(x3[pl.ds(_G1, _R1), :], w1k[1],
                        preferred_element_type=f32)
              + jnp.dot(x3[pl.ds(2 * _G1, _R1), :], w1k[2],
                        preferred_element_type=f32))
        v1h = jnp.maximum(z1, 0.0).astype(bf16)
        ca[...] = jnp.concatenate(
            [v1h, jnp.concatenate([v1h[1:], v1h[:1]], axis=0)], axis=1)
        cb[...] = jnp.concatenate(
            [v1h, jnp.concatenate([v1h[_G1:], v1h[:_G1]], axis=0)], axis=1)

    def stage_conv23(g):
        ca, cb, t3 = S[g][1], S[g][2], S[g][3]
        d0 = jnp.dot(ca[pl.ds(0, _R2), :], w2p[0], preferred_element_type=f32)
        d1 = jnp.dot(ca[pl.ds(_G1, _R2), :], w2p[1],
                     preferred_element_type=f32)
        d2 = jnp.dot(ca[pl.ds(2 * _G1, _R2), :], w2p[2],
                     preferred_element_type=f32)
        d3 = jnp.dot(cb[pl.ds(2, _R2), :], w2p[3], preferred_element_type=f32)
        d4 = jnp.dot(cb[pl.ds(_G1 + 2, _R2), :], w2p[4],
                     preferred_element_type=f32)
        z2 = ((d0 + d1) + (d2 + d3)) + d4
        y2 = jnp.maximum(z2 + c2[...], 0.0).astype(bf16)
        z3 = jnp.dot(y2, w3c[0], preferred_element_type=f32)
        t3[...] = jnp.maximum(z3 + c3[...], 0.0) * s3[...] + b3[...]

    def stage_pool(g):
        t3, ts, pa, pb = S[g][3], S[g][4], S[g][5], S[g][6]
        ts[pl.ds(0, _R2 - 1), :] = jnp.maximum(t3[pl.ds(0, _R2 - 1), :],
                                               t3[pl.ds(1, _R2 - 1), :])
        prev = None
        for i in range(_BANDS):
            base = 2 * i * _G1
            p = jnp.maximum(ts[pl.ds(base, _BANDS, 2), :],
                            ts[pl.ds(base + _G1, _BANDS, 2), :]).astype(bf16)
            pa[pl.ds(_G2 * i, _BANDS), :] = jnp.concatenate(
                [p, jnp.concatenate([p[1:], p[:1]], axis=0)], axis=1)
            if prev is not None:
                pb[pl.ds(_G2 * (i - 1), _BANDS), :] = jnp.concatenate(
                    [prev, p], axis=1)
            prev = p
        pb[pl.ds(_G2 * (_BANDS - 1), _BANDS), :] = jnp.concatenate(
            [prev, prev], axis=1)

    def _acc(zs, g, d):
        zs[g] = d if zs[g] is None else zs[g] + d

    def stage_conv4_all():
        zs = [None] * _GRP
        for t, (buf, off) in enumerate(
                ((5, 0), (5, _G2), (5, 2 * _G2), (6, 2), (6, _G2 + 2))):
            wt = w4p[t]
            for g in range(_GRP):
                _acc(zs, g, jnp.dot(S[g][buf][pl.ds(off, _R4), :], wt,
                                    preferred_element_type=f32))
        for g in range(_GRP):
            S[g][7][...] = jnp.maximum(zs[g], 0.0).astype(bf16)

    def stage_conv5_all():
        zs = [None] * _GRP
        for t in range(9):
            wt = w5[t]
            off = (t // 3) * _G2 + t % 3
            for g in range(_GRP):
                _acc(zs, g, jnp.dot(S[g][7][pl.ds(off, _R5), :], wt,
                                    preferred_element_type=f32))
        for g in range(_GRP):
            S[g][8][...] = jnp.maximum(zs[g] + c5[...], 0.0).astype(bf16)

    def stage_conv6_head_all():
        zs = [None] * _GRP
        for t in range(9):
            wt = w6[t]
            off = (t // 3) * _G2 + t % 3
            for g in range(_GRP):
                _acc(zs, g, jnp.dot(S[g][8][pl.ds(off, _R6), :], wt,
                                    preferred_element_type=f32))
        for g in range(_GRP):
            tf = S[g][9]
            r6 = jnp.maximum(zs[g] + c6[...], 0.0)
            ssum = None
            for i in range(_HEXT):
                band = jnp.sum(r6[_G2 * i:_G2 * i + _HEXT, :],
                               axis=0, keepdims=True)
                ssum = band if ssum is None else ssum + band
            pooled = ssum * (1.0 / (_HEXT * _HEXT))
            for b in range(_NB):
                tf[pl.ds(b, 1), :] = pooled[:, 16 * b:16 * (b + 1)]
            logits = jnp.dot(tf[...], wh[...],
                             preferred_element_type=f32) + bh[...]
            zc = logits - jnp.max(logits, axis=-1, keepdims=True)
            o_ref[pl.ds(g * _NB, _NB), :] = (
                zc - jnp.log(jnp.sum(jnp.exp(zc), axis=-1, keepdims=True)))

    for stage in (stage_x3, stage_conv1, stage_conv23, stage_pool):
        for g in range(_GRP):
            stage(g)
    stage_conv4_all()
    stage_conv5_all()
    stage_conv6_head_all()


def _fold(gamma, beta, mean, var):
    scale = gamma / jnp.sqrt(var + _EPS)
    return scale, beta - mean * scale


def _block_taps(w, scale_in=None):
    w = w.astype(jnp.float32)
    if scale_in is not None:
        w = w * scale_in.astype(jnp.float32)[None, :, None, None]
    cout, cin = w.shape[0], w.shape[1]
    taps = jnp.transpose(w, (2, 3, 1, 0)).reshape(-1, cin, cout)
    eye = jnp.eye(_NB, dtype=jnp.float32)
    wbd = jnp.einsum("ab,tio->taibo", eye, taps)
    return wbd.reshape(taps.shape[0], _NB * cin, _NB * cout)


def _pair_w(taps):
    k, n = taps.shape[1], taps.shape[2]
    z = jnp.zeros((k, n), jnp.float32)
    cat = lambda a, b: jnp.concatenate([a, b], axis=0)
    return jnp.stack([cat(taps[0], taps[1]), cat(taps[3], taps[4]),
                      cat(taps[6], taps[7]), cat(taps[2], taps[5]),
                      cat(z, taps[8])])


def _shift_bias(w, shift_in):
    c = jnp.einsum("oikl,i->o", w.astype(jnp.float32),
                   shift_in.astype(jnp.float32))
    return jnp.tile(c, _NB).reshape(1, -1)


def kernel(x, w1_1, w1_2, w1_3, w2_1, w2_2, w2_3, w3,
           bn1_1_gamma, bn1_1_beta, bn1_1_mean, bn1_1_var,
           bn1_2_gamma, bn1_2_beta, bn1_2_mean, bn1_2_var,
           bn1_3_gamma, bn1_3_beta, bn1_3_mean, bn1_3_var,
           bn2_1_gamma, bn2_1_beta, bn2_1_mean, bn2_1_var,
           bn2_2_gamma, bn2_2_beta, bn2_2_mean, bn2_2_var,
           bn2_3_gamma, bn2_3_beta, bn2_3_mean, bn2_3_var):
    s1, sh1 = _fold(bn1_1_gamma, bn1_1_beta, bn1_1_mean, bn1_1_var)
    s2, sh2 = _fold(bn1_2_gamma, bn1_2_beta, bn1_2_mean, bn1_2_var)
    s3, sh3 = _fold(bn1_3_gamma, bn1_3_beta, bn1_3_mean, bn1_3_var)
    s4, sh4 = _fold(bn2_1_gamma, bn2_1_beta, bn2_1_mean, bn2_1_var)
    s5, sh5 = _fold(bn2_2_gamma, bn2_2_beta, bn2_2_mean, bn2_2_var)
    s6, sh6 = _fold(bn2_3_gamma, bn2_3_beta, bn2_3_mean, bn2_3_var)

    n = x.shape[0]
    per = _NB * _GRP
    steps = -(-n // per)
    n_pad = steps * per
    xf = x.astype(jnp.float32).reshape(n, _S1)
    if n_pad != n:
        xf = jnp.concatenate(
            [xf, jnp.zeros((n_pad - n, _S1), jnp.float32)], axis=0)
    xs = xf.reshape(steps, per, _S1).astype(jnp.bfloat16)

    bf16 = jnp.bfloat16
    t1aps = _block_taps(w1_1)
    kpad = jnp.zeros((_L - 48, _L), jnp.float32)
    w1k = jnp.stack([jnp.concatenate([t1aps[3 * kh], t1aps[3 * kh + 1],
                                      t1aps[3 * kh + 2], kpad], axis=0)
                     for kh in range(3)]).astype(bf16)
    w2p = _pair_w(_block_taps(w1_2, scale_in=s1)).astype(bf16)
    c2 = _shift_bias(w1_2, sh1)
    w3b = _block_taps(w1_3, scale_in=s2).astype(bf16)
    c3 = _shift_bias(w1_3, sh2)
    w21p = jnp.concatenate(
        [w2_1.astype(jnp.float32),
         jnp.zeros((4,) + w2_1.shape[1:], jnp.float32)], axis=0)
    w4p = _pair_w(_block_taps(w21p)).astype(bf16)
    w22p = jnp.concatenate(
        [w2_2.astype(jnp.float32),
         jnp.zeros((w2_2.shape[0], 4) + w2_2.shape[2:], jnp.float32)], axis=1)
    s4p = jnp.concatenate([s4, jnp.ones((4,), jnp.float32)])
    w5b = _block_taps(w22p, scale_in=s4p).astype(bf16)
    c5 = _shift_bias(w2_2, sh4)
    w6b = _block_taps(w2_3, scale_in=s5).astype(bf16)
    c6 = _shift_bias(w2_3, sh5)
    s3t = jnp.tile(s3, _NB).reshape(1, -1)
    b3t = jnp.tile(sh3, _NB).reshape(1, -1)
    w3f = jnp.transpose(w3[:, :, 0, 0]).astype(jnp.float32)
    wh = w3f * s6.astype(jnp.float32)[:, None]
    bh = (sh6.astype(jnp.float32) @ w3f).reshape(1, 10)

    full = lambda *shape: pl.BlockSpec(shape, lambda s: (0,) * len(shape))
    in_specs = [
        pl.BlockSpec((None, _NB * _GRP, _S1), lambda s: (s, 0, 0)),
        full(3, _L, _L),
        full(5, 256, 256),
        full(1, 256),
        full(1, 256, _L),
        full(1, _L),
        full(1, _L),
        full(1, _L),
        full(5, 256, 256),
        full(9, 256, 256),
        full(1, 256),
        full(9, 256, 256),
        full(1, 256),
        full(16, 10),
        full(1, 10),
    ]
    args = (xs, w1k, w2p, c2, w3b, c3, s3t, b3t,
            w4p, w5b, c5, w6b, c6, wh, bh)

    group_scr = [
        pltpu.VMEM((_S1, _L), jnp.bfloat16),
        pltpu.VMEM((_R1, 2 * _L), jnp.bfloat16),
        pltpu.VMEM((_R1, 2 * _L), jnp.bfloat16),
        pltpu.VMEM((_R2, _L), jnp.float32),
        pltpu.VMEM((_R2, _L), jnp.float32),
        pltpu.VMEM((_RP, 2 * _L), jnp.bfloat16),
        pltpu.VMEM((_RP, 2 * _L), jnp.bfloat16),
        pltpu.VMEM((_R4, _NB * 16), jnp.bfloat16),
        pltpu.VMEM((_R5, _NB * 16), jnp.bfloat16),
        pltpu.VMEM((_NB, 16), jnp.float32),
    ]
    out = pl.pallas_call(
        _net_kernel,
        out_shape=jax.ShapeDtypeStruct((steps, _NB * _GRP, 10), jnp.float32),
        grid=(steps,),
        in_specs=in_specs,
        out_specs=pl.BlockSpec((None, _NB * _GRP, 10), lambda s: (s, 0, 0)),
        scratch_shapes=group_scr * _GRP,
        compiler_params=pltpu.CompilerParams(
            dimension_semantics=("parallel",),
            vmem_limit_bytes=48 * 1024 * 1024,
        ),
    )(*args)
    return out.reshape(n_pad, 10)[:n]
